# Initial kernel scaffold; baseline (speedup 1.0000x reference)
import jax
import jax.numpy as jnp
from jax.experimental import pallas as pl


def kernel(x, edge_index, W1, b1, W2, b2):
    raise NotImplementedError("write your pallas kernel here")


# speedup vs baseline: 1.5146x; 1.5146x over previous
import functools

import jax
import jax.numpy as jnp
from jax.experimental import pallas as pl
from jax.experimental.pallas import tpu as pltpu

N_NODES = 10000
ROW_BLK = 1000


def _mm_kernel(x_ref, w_ref, o_ref):
    o_ref[...] = jnp.dot(x_ref[...], w_ref[...],
                         preferred_element_type=jnp.float32)


def _matmul(x, w):
    n, k = x.shape
    _, m = w.shape
    grid = n // ROW_BLK
    return pl.pallas_call(
        _mm_kernel,
        grid=(grid,),
        in_specs=[
            pl.BlockSpec((ROW_BLK, k), lambda i: (i, 0)),
            pl.BlockSpec((k, m), lambda i: (0, 0)),
        ],
        out_specs=pl.BlockSpec((ROW_BLK, m), lambda i: (i, 0)),
        out_shape=jax.ShapeDtypeStruct((n, m), jnp.float32),
    )(x, w)


def _relu_mm_kernel(x_ref, b_ref, w_ref, o_ref):
    h = jnp.maximum(x_ref[...] + b_ref[...], 0.0)
    o_ref[...] = jnp.dot(h, w_ref[...], preferred_element_type=jnp.float32)


def _relu_matmul(x, b, w):
    n, k = x.shape
    _, m = w.shape
    grid = n // ROW_BLK
    return pl.pallas_call(
        _relu_mm_kernel,
        grid=(grid,),
        in_specs=[
            pl.BlockSpec((ROW_BLK, k), lambda i: (i, 0)),
            pl.BlockSpec((1, k), lambda i: (0, 0)),
            pl.BlockSpec((k, m), lambda i: (0, 0)),
        ],
        out_specs=pl.BlockSpec((ROW_BLK, m), lambda i: (i, 0)),
        out_shape=jax.ShapeDtypeStruct((n, m), jnp.float32),
    )(x, b.reshape(1, k), w)


def _tail_kernel(x_ref, h_ref, b_ref, o_ref):
    o_ref[...] = jnp.tanh(x_ref[...] + h_ref[...] + b_ref[...])


def _tail(x128, h, b):
    n, m = h.shape
    grid = n // ROW_BLK
    return pl.pallas_call(
        _tail_kernel,
        grid=(grid,),
        in_specs=[
            pl.BlockSpec((ROW_BLK, m), lambda i: (i, 0)),
            pl.BlockSpec((ROW_BLK, m), lambda i: (i, 0)),
            pl.BlockSpec((1, m), lambda i: (0, 0)),
        ],
        out_specs=pl.BlockSpec((ROW_BLK, m), lambda i: (i, 0)),
        out_shape=jax.ShapeDtypeStruct((n, m), jnp.float32),
    )(x128, h, b.reshape(1, m))


def _seg_max(h, src, dst):
    msgs = jnp.take(h, src, axis=0)
    return jax.ops.segment_max(msgs, dst, num_segments=N_NODES,
                               indices_are_sorted=False)


def kernel(x, edge_index, W1, b1, W2, b2):
    src = edge_index[0]
    dst = edge_index[1]
    h1 = _matmul(x, W1)
    m1 = jnp.maximum(_seg_max(h1, src, dst), h1)
    h2 = _relu_matmul(m1, b1, W2)
    m2 = jnp.maximum(_seg_max(h2, src, dst), h2)
    return _tail(x[:, :-2], m2, b2)


# speedup vs baseline: 4.9201x; 3.2485x over previous
import functools

import jax
import jax.numpy as jnp
from jax import lax
from jax.experimental import pallas as pl
from jax.experimental.pallas import tpu as pltpu
from jax.experimental.pallas import tpu_sc as plsc

N_NODES = 10000
N_EDGES = 320000
NW = 32
RPT = 320
NPAD = NW * RPT
EPW = N_EDGES // NW
CAP = 512
QTOT = NW * NW * CAP
DUMMY = RPT

ROW_BLK = 1024

_mesh = functools.partial(
    plsc.VectorSubcoreMesh, core_axis_name="c", subcore_axis_name="s",
    num_cores=2, num_subcores=16)

_sc_params = pltpu.CompilerParams(needs_layout_passes=False)


def _iota16():
    return lax.iota(jnp.int32, 16)


def _vgather(v, idx):
    return v.at[idx].get(mode="promise_in_bounds")


def _lane(v, e):
    return jnp.max(jnp.where(_iota16() == e, v, 0))


def _wid():
    return lax.axis_index("s") * 2 + lax.axis_index("c")


def _bucket_body(edge_hbm, qsrc_hbm, qdst_hbm, counts_hbm,
                 es_v, ed_v, qs_v, qd_v, cnt_v, ccnt_v, sem):
    w = _wid()
    pltpu.async_copy(edge_hbm.at[pl.ds(w * EPW, EPW)], es_v, sem).wait()
    pltpu.async_copy(edge_hbm.at[pl.ds(N_EDGES + w * EPW, EPW)], ed_v,
                     sem).wait()

    iota = _iota16()
    zeros = jnp.zeros((16,), jnp.int32)
    dummyv = jnp.full((16,), DUMMY, jnp.int32)

    cnt_v[pl.ds(0, 16)] = zeros
    cnt_v[pl.ds(16, 16)] = zeros

    def prefill(r, carry):
        base = r * 256
        for k in range(16):
            qd_v[pl.ds(base + k * 16, 16)] = dummyv
            qs_v[pl.ds(base + k * 16, 16)] = zeros
        return carry
    lax.fori_loop(0, NW * CAP // 256, prefill, 0)

    def body(i, carry):
        s = es_v[pl.ds(i * 16, 16)]
        d = ed_v[pl.ds(i * 16, 16)]
        b = ((d >> 6) * 205) >> 10
        bs, perm = plsc.sort_key_val(b, iota)
        ss = _vgather(s, perm)
        ds = _vgather(d, perm)
        prev = _vgather(bs, jnp.maximum(iota - 1, 0))
        rs = (iota == 0) | (bs != prev)
        sidx = plsc.cummax(jnp.where(rs, iota, 0))
        rank = iota - sidx
        base = plsc.load_gather(cnt_v, [bs])
        pos = base + rank
        nxt = _vgather(rs.astype(jnp.int32), jnp.minimum(iota + 1, 15))
        ls = (iota == 15) | ((iota < 15) & (nxt == 1))
        plsc.store_scatter(qs_v, [bs * CAP + pos], ss)
        plsc.store_scatter(qd_v, [bs * CAP + pos], ds - bs * RPT)
        plsc.store_scatter(cnt_v, [bs], pos + 1, mask=ls)
        return carry
    lax.fori_loop(0, EPW // 16, body, 0)

    ccnt_v[pl.ds(0, 16)] = (cnt_v[pl.ds(0, 16)] + 15) >> 4
    ccnt_v[pl.ds(16, 16)] = (cnt_v[pl.ds(16, 16)] + 15) >> 4

    pltpu.async_copy(qs_v, qsrc_hbm.at[pl.ds(w * NW * CAP, NW * CAP)],
                     sem).wait()
    pltpu.async_copy(qd_v, qdst_hbm.at[pl.ds(w * NW * CAP, NW * CAP)],
                     sem).wait()
    pltpu.async_copy(ccnt_v, counts_hbm.at[pl.ds(w * NW, NW)], sem).wait()


@functools.partial(
    pl.kernel,
    out_type=(jax.ShapeDtypeStruct((QTOT,), jnp.int32),
              jax.ShapeDtypeStruct((QTOT,), jnp.int32),
              jax.ShapeDtypeStruct((NW * NW,), jnp.int32)),
    mesh=_mesh(),
    scratch_types=(pltpu.VMEM((EPW,), jnp.int32),
                   pltpu.VMEM((EPW,), jnp.int32),
                   pltpu.VMEM((NW * CAP,), jnp.int32),
                   pltpu.VMEM((NW * CAP,), jnp.int32),
                   pltpu.VMEM((NW,), jnp.int32),
                   pltpu.VMEM((NW,), jnp.int32),
                   pltpu.SemaphoreType.DMA),
    compiler_params=_sc_params,
)
def _bucket_edges(edge_hbm, qsrc_hbm, qdst_hbm, counts_hbm, *rest):
    _bucket_body(edge_hbm, qsrc_hbm, qdst_hbm, counts_hbm, *rest)


def _segmax_body(h_hbm, qsrc_hbm, qdst_hbm, counts_hbm, out_hbm,
                 acc_v, qs_v, qd_v, rows_v, cnt_v, sem0, sem1):
    t = _wid()
    pltpu.async_copy(h_hbm.at[pl.ds(t * RPT, RPT)], acc_v.at[pl.ds(0, RPT)],
                     sem0).wait()
    pltpu.async_copy(counts_hbm, cnt_v, sem0).wait()

    iota = _iota16()

    def w_body(w, carry):
        qoff = (w * NW + t) * CAP
        pltpu.async_copy(qsrc_hbm.at[pl.ds(qoff, CAP)], qs_v, sem0)
        pltpu.async_copy(qdst_hbm.at[pl.ds(qoff, CAP)], qd_v, sem1)
        pltpu.make_async_copy(qsrc_hbm.at[pl.ds(qoff, CAP)], qs_v, sem0).wait()
        pltpu.make_async_copy(qdst_hbm.at[pl.ds(qoff, CAP)], qd_v, sem1).wait()
        cidx = w * NW + t
        crow = cnt_v[pl.ds((cidx // 16) * 16, 16)]
        ncha = jnp.max(jnp.where(iota == cidx - (cidx // 16) * 16, crow, 0))

        @pl.when(ncha > 0)
        def _():
            pltpu.async_copy(h_hbm.at[qs_v.at[pl.ds(0, 16)]],
                             rows_v.at[pl.ds(0, 16)], sem0)

        def g_body(g, carry):
            nxt = g + 1

            @pl.when(nxt < ncha)
            def _():
                @pl.when(lax.rem(nxt, 2) == 0)
                def _():
                    pltpu.async_copy(h_hbm.at[qs_v.at[pl.ds(nxt * 16, 16)]],
                                     rows_v.at[pl.ds(0, 16)], sem0)

                @pl.when(lax.rem(nxt, 2) == 1)
                def _():
                    pltpu.async_copy(h_hbm.at[qs_v.at[pl.ds(nxt * 16, 16)]],
                                     rows_v.at[pl.ds(16, 16)], sem1)

            par = lax.rem(g, 2)

            @pl.when(par == 0)
            def _():
                pltpu.make_async_copy(h_hbm.at[qs_v.at[pl.ds(0, 16)]],
                                      rows_v.at[pl.ds(0, 16)], sem0).wait()

            @pl.when(par == 1)
            def _():
                pltpu.make_async_copy(h_hbm.at[qs_v.at[pl.ds(0, 16)]],
                                      rows_v.at[pl.ds(16, 16)], sem1).wait()

            dvec = qd_v[pl.ds(g * 16, 16)]
            rbase = par * 16
            for e in range(16):
                d_e = _lane(dvec, e)
                r = rbase + e
                for j in range(8):
                    sl = pl.ds(j * 16, 16)
                    acc_v[d_e, sl] = jnp.maximum(acc_v[d_e, sl], rows_v[r, sl])
            return carry

        lax.fori_loop(0, ncha, g_body, 0)
        return carry

    lax.fori_loop(0, NW, w_body, 0)
    pltpu.async_copy(acc_v.at[pl.ds(0, RPT)], out_hbm.at[pl.ds(t * RPT, RPT)],
                     sem0).wait()


@functools.partial(
    pl.kernel,
    out_type=jax.ShapeDtypeStruct((NPAD, 128), jnp.float32),
    mesh=_mesh(),
    scratch_types=(pltpu.VMEM((RPT + 8, 128), jnp.float32),
                   pltpu.VMEM((CAP,), jnp.int32),
                   pltpu.VMEM((CAP,), jnp.int32),
                   pltpu.VMEM((32, 128), jnp.float32),
                   pltpu.VMEM((NW * NW,), jnp.int32),
                   pltpu.SemaphoreType.DMA,
                   pltpu.SemaphoreType.DMA),
    compiler_params=_sc_params,
)
def _segmax(h_hbm, qsrc_hbm, qdst_hbm, counts_hbm, out_hbm, *rest):
    _segmax_body(h_hbm, qsrc_hbm, qdst_hbm, counts_hbm, out_hbm, *rest)


def _mm1_kernel(x_ref, w_ref, o_ref):
    o_ref[...] = jnp.dot(x_ref[...], w_ref[...],
                         preferred_element_type=jnp.float32)


def _matmul1(x, w):
    n, k = x.shape
    _, m = w.shape
    grid = pl.cdiv(n, ROW_BLK)
    return pl.pallas_call(
        _mm1_kernel,
        grid=(grid,),
        in_specs=[
            pl.BlockSpec((ROW_BLK, k), lambda i: (i, 0)),
            pl.BlockSpec((k, m), lambda i: (0, 0)),
        ],
        out_specs=pl.BlockSpec((ROW_BLK, m), lambda i: (i, 0)),
        out_shape=jax.ShapeDtypeStruct((n, m), jnp.float32),
    )(x, w)


def _relu_mm_kernel(x_ref, b_ref, w_ref, o_ref):
    h = jnp.maximum(x_ref[...] + b_ref[...], 0.0)
    o_ref[...] = jnp.dot(h, w_ref[...], preferred_element_type=jnp.float32)


def _relu_matmul(x, b, w):
    n, k = x.shape
    _, m = w.shape
    grid = pl.cdiv(n, ROW_BLK)
    return pl.pallas_call(
        _relu_mm_kernel,
        grid=(grid,),
        in_specs=[
            pl.BlockSpec((ROW_BLK, k), lambda i: (i, 0)),
            pl.BlockSpec((1, k), lambda i: (0, 0)),
            pl.BlockSpec((k, m), lambda i: (0, 0)),
        ],
        out_specs=pl.BlockSpec((ROW_BLK, m), lambda i: (i, 0)),
        out_shape=jax.ShapeDtypeStruct((n, m), jnp.float32),
    )(x, b.reshape(1, k), w)


def _tail_kernel(x_ref, h_ref, b_ref, o_ref):
    o_ref[...] = jnp.tanh(x_ref[:, :128] + h_ref[...] + b_ref[...])


def _tail(x, h, b):
    n = N_NODES
    m = 128
    blk = 1000
    return pl.pallas_call(
        _tail_kernel,
        grid=(n // blk,),
        in_specs=[
            pl.BlockSpec((blk, x.shape[1]), lambda i: (i, 0)),
            pl.BlockSpec((blk, m), lambda i: (i, 0)),
            pl.BlockSpec((1, m), lambda i: (0, 0)),
        ],
        out_specs=pl.BlockSpec((blk, m), lambda i: (i, 0)),
        out_shape=jax.ShapeDtypeStruct((n, m), jnp.float32),
    )(x, h, b.reshape(1, m))


def kernel(x, edge_index, W1, b1, W2, b2):
    xp = jnp.pad(x, ((0, NPAD - N_NODES), (0, 0)))
    qsrc, qdst, counts = _bucket_edges(edge_index.reshape(-1))
    h1 = _matmul1(xp, W1)
    m1 = _segmax(h1, qsrc, qdst, counts)
    h2 = _relu_matmul(m1, b1, W2)
    m2 = _segmax(h2, qsrc, qdst, counts)
    return _tail(x, m2, b2)


# speedup vs baseline: 6.3360x; 1.2878x over previous
import functools

import jax
import jax.numpy as jnp
from jax import lax
from jax.experimental import pallas as pl
from jax.experimental.pallas import tpu as pltpu
from jax.experimental.pallas import tpu_sc as plsc

N_NODES = 10000
N_EDGES = 320000
NW = 32
RPT = 320
NPAD = NW * RPT
EPW = N_EDGES // NW
CAP = 512
QTOT = NW * NW * CAP
DUMMY = RPT

ROW_BLK = 1024

_mesh = functools.partial(
    plsc.VectorSubcoreMesh, core_axis_name="c", subcore_axis_name="s",
    num_cores=2, num_subcores=16)

_sc_params = pltpu.CompilerParams(needs_layout_passes=False)


def _iota16():
    return lax.iota(jnp.int32, 16)


def _vgather(v, idx):
    return v.at[idx].get(mode="promise_in_bounds")


def _lane(v, e):
    return jnp.max(jnp.where(_iota16() == e, v, 0))


def _wid():
    return lax.axis_index("s") * 2 + lax.axis_index("c")


def _bucket_body(edge_hbm, qsrc_hbm, qdst_hbm, counts_hbm,
                 es_v, ed_v, qs_v, qd_v, cnt_v, ccnt_v, sem):
    w = _wid()
    pltpu.async_copy(edge_hbm.at[pl.ds(w * EPW, EPW)], es_v, sem).wait()
    pltpu.async_copy(edge_hbm.at[pl.ds(N_EDGES + w * EPW, EPW)], ed_v,
                     sem).wait()

    iota = _iota16()
    zeros = jnp.zeros((16,), jnp.int32)
    dummyv = jnp.full((16,), DUMMY, jnp.int32)

    cnt_v[pl.ds(0, 16)] = zeros
    cnt_v[pl.ds(16, 16)] = zeros

    def prefill(r, carry):
        base = r * 256
        for k in range(16):
            qd_v[pl.ds(base + k * 16, 16)] = dummyv
            qs_v[pl.ds(base + k * 16, 16)] = zeros
        return carry
    lax.fori_loop(0, NW * CAP // 256, prefill, 0)

    def body(i, carry):
        s = es_v[pl.ds(i * 16, 16)]
        d = ed_v[pl.ds(i * 16, 16)]
        b = ((d >> 6) * 205) >> 10
        bs, perm = plsc.sort_key_val(b, iota)
        ss = _vgather(s, perm)
        ds = _vgather(d, perm)
        prev = _vgather(bs, jnp.maximum(iota - 1, 0))
        rs = (iota == 0) | (bs != prev)
        sidx = plsc.cummax(jnp.where(rs, iota, 0))
        rank = iota - sidx
        base = plsc.load_gather(cnt_v, [bs])
        pos = base + rank
        nxt = _vgather(rs.astype(jnp.int32), jnp.minimum(iota + 1, 15))
        ls = (iota == 15) | ((iota < 15) & (nxt == 1))
        plsc.store_scatter(qs_v, [bs * CAP + pos], ss)
        plsc.store_scatter(qd_v, [bs * CAP + pos], ds - bs * RPT)
        plsc.store_scatter(cnt_v, [bs], pos + 1, mask=ls)
        return carry
    lax.fori_loop(0, EPW // 16, body, 0)

    ccnt_v[pl.ds(0, 16)] = (cnt_v[pl.ds(0, 16)] + 15) >> 4
    ccnt_v[pl.ds(16, 16)] = (cnt_v[pl.ds(16, 16)] + 15) >> 4

    pltpu.async_copy(qs_v, qsrc_hbm.at[pl.ds(w * NW * CAP, NW * CAP)],
                     sem).wait()
    pltpu.async_copy(qd_v, qdst_hbm.at[pl.ds(w * NW * CAP, NW * CAP)],
                     sem).wait()
    pltpu.async_copy(ccnt_v, counts_hbm.at[pl.ds(w * NW, NW)], sem).wait()


@functools.partial(
    pl.kernel,
    out_type=(jax.ShapeDtypeStruct((QTOT,), jnp.int32),
              jax.ShapeDtypeStruct((QTOT,), jnp.int32),
              jax.ShapeDtypeStruct((NW * NW,), jnp.int32)),
    mesh=_mesh(),
    scratch_types=(pltpu.VMEM((EPW,), jnp.int32),
                   pltpu.VMEM((EPW,), jnp.int32),
                   pltpu.VMEM((NW * CAP,), jnp.int32),
                   pltpu.VMEM((NW * CAP,), jnp.int32),
                   pltpu.VMEM((NW,), jnp.int32),
                   pltpu.VMEM((NW,), jnp.int32),
                   pltpu.SemaphoreType.DMA),
    compiler_params=_sc_params,
)
def _bucket_edges(edge_hbm, qsrc_hbm, qdst_hbm, counts_hbm, *rest):
    _bucket_body(edge_hbm, qsrc_hbm, qdst_hbm, counts_hbm, *rest)


SCAP = 20480
GCAP = 2560
WG = 24
WROWS = WG * 8


def _merge_sort_body(qsrc_hbm, qdst_hbm, counts_hbm,
                     ssrc_hbm, gd_hbm, tot_hbm,
                     qs_v, qd_v, bins_v, off_v, ng_v, cnt2_v,
                     ssrc_v, gd_v, tot_v, sem):
    t = _wid()
    iota = _iota16()
    zeros = jnp.zeros((16,), jnp.int32)

    def stage(w, carry):
        qoff = (w * NW + t) * CAP
        pltpu.async_copy(qsrc_hbm.at[pl.ds(qoff, CAP)],
                         qs_v.at[pl.ds(w * CAP, CAP)], sem)
        pltpu.make_async_copy(qsrc_hbm.at[pl.ds(qoff, CAP)],
                              qs_v.at[pl.ds(w * CAP, CAP)], sem).wait()
        pltpu.async_copy(qdst_hbm.at[pl.ds(qoff, CAP)],
                         qd_v.at[pl.ds(w * CAP, CAP)], sem)
        pltpu.make_async_copy(qdst_hbm.at[pl.ds(qoff, CAP)],
                              qd_v.at[pl.ds(w * CAP, CAP)], sem).wait()
        return carry
    lax.fori_loop(0, NW, stage, 0)

    for k in range(328 // 8 // 2):
        bins_v[pl.ds(k * 16, 16)] = zeros
        cnt2_v[pl.ds(k * 16, 16)] = zeros
    bins_v[pl.ds(328 - 16, 16)] = zeros
    cnt2_v[pl.ds(328 - 16, 16)] = zeros

    neg1 = jnp.full((16,), -1, jnp.int32)
    dum = jnp.full((16,), DUMMY, jnp.int32)

    def pre1(i, carry):
        ssrc_v[pl.ds(i * 16, 16)] = neg1
        return carry
    lax.fori_loop(0, SCAP // 16, pre1, 0)

    def pre2(i, carry):
        gd_v[pl.ds(i * 16, 16)] = dum
        return carry
    lax.fori_loop(0, GCAP // 16, pre2, 0)

    def hist(i, carry):
        d = qd_v[pl.ds(i * 16, 16)]
        ds, _ = plsc.sort_key_val(d, iota)
        prev = _vgather(ds, jnp.maximum(iota - 1, 0))
        rs = (iota == 0) | (ds != prev)
        sidx = plsc.cummax(jnp.where(rs, iota, 0))
        rank = iota - sidx
        nxt = _vgather(rs.astype(jnp.int32), jnp.minimum(iota + 1, 15))
        ls = (iota == 15) | ((iota < 15) & (nxt == 1))
        base = plsc.load_gather(bins_v, [ds])
        plsc.store_scatter(bins_v, [ds], base + rank + 1, mask=ls)
        return carry
    lax.fori_loop(0, NW * CAP // 16, hist, 0)

    def prefix(v, carry):
        b = bins_v[pl.ds(v * 16, 16)]
        pad8 = (b + 7) & ~7
        cs = plsc.cumsum(pad8)
        off_v[pl.ds(v * 16, 16)] = carry + cs - pad8
        ng_v[pl.ds(v * 16, 16)] = pad8 >> 3
        return carry + cs[15]
    carry = lax.fori_loop(0, 320 // 16, prefix, 0)

    def gdfill(v, carry):
        offv = off_v[pl.ds(v * 16, 16)]
        ngv = ng_v[pl.ds(v * 16, 16)]
        for lane in range(16):
            goff = offv[lane] >> 3
            n = ngv[lane]
            dval = jnp.full((16,), v * 16 + lane, jnp.int32)

            def put(k, c):
                plsc.store_scatter(gd_v, [jnp.where(iota == 0, goff + k, 0)],
                                   dval, mask=(iota == 0))
                return c
            lax.fori_loop(0, n, put, 0)
        return carry
    lax.fori_loop(0, 20, gdfill, 0)

    def place(i, carry):
        s = qs_v[pl.ds(i * 16, 16)]
        d = qd_v[pl.ds(i * 16, 16)]
        ds, perm = plsc.sort_key_val(d, iota)
        ss = _vgather(s, perm)
        prev = _vgather(ds, jnp.maximum(iota - 1, 0))
        rs = (iota == 0) | (ds != prev)
        sidx = plsc.cummax(jnp.where(rs, iota, 0))
        rank = iota - sidx
        nxt = _vgather(rs.astype(jnp.int32), jnp.minimum(iota + 1, 15))
        ls = (iota == 15) | ((iota < 15) & (nxt == 1))
        valid = ds != DUMMY
        base = plsc.load_gather(cnt2_v, [ds])
        tgt = plsc.load_gather(off_v, [jnp.minimum(ds, 319)])
        plsc.store_scatter(ssrc_v, [jnp.minimum(tgt + base + rank, SCAP - 1)],
                           ss, mask=valid)
        plsc.store_scatter(cnt2_v, [ds], base + rank + 1, mask=ls)
        return carry
    lax.fori_loop(0, NW * CAP // 16, place, 0)

    g_tot = carry >> 3
    nwin = ((g_tot + WG - 1) * 2731) >> 16
    tot_v[pl.ds(0, 16)] = jnp.where(iota == 0, nwin, 0)

    pltpu.async_copy(ssrc_v, ssrc_hbm.at[pl.ds(t * SCAP, SCAP)], sem).wait()
    pltpu.async_copy(gd_v, gd_hbm.at[pl.ds(t * GCAP, GCAP)], sem).wait()
    pltpu.async_copy(tot_v, tot_hbm.at[pl.ds(t * 16, 16)], sem).wait()


@functools.partial(
    pl.kernel,
    out_type=(jax.ShapeDtypeStruct((NW * SCAP,), jnp.int32),
              jax.ShapeDtypeStruct((NW * GCAP,), jnp.int32),
              jax.ShapeDtypeStruct((NW * 16,), jnp.int32)),
    mesh=_mesh(),
    scratch_types=(pltpu.VMEM((NW * CAP,), jnp.int32),
                   pltpu.VMEM((NW * CAP,), jnp.int32),
                   pltpu.VMEM((328,), jnp.int32),
                   pltpu.VMEM((328,), jnp.int32),
                   pltpu.VMEM((320,), jnp.int32),
                   pltpu.VMEM((328,), jnp.int32),
                   pltpu.VMEM((SCAP,), jnp.int32),
                   pltpu.VMEM((GCAP,), jnp.int32),
                   pltpu.VMEM((16,), jnp.int32),
                   pltpu.SemaphoreType.DMA),
    compiler_params=_sc_params,
)
def _merge_sort(qsrc_hbm, qdst_hbm, counts_hbm, ssrc_hbm, gd_hbm, tot_hbm,
                *rest):
    _merge_sort_body(qsrc_hbm, qdst_hbm, counts_hbm, ssrc_hbm, gd_hbm,
                     tot_hbm, *rest)


def _segmax2_body(h_hbm, ssrc_hbm, gd_hbm, tot_hbm, out_hbm,
                  acc_v, ssrc_v, gd_v, rows_v, tot_v, sem0, sem1):
    t = _wid()
    iota = _iota16()
    pltpu.async_copy(h_hbm.at[pl.ds(t * RPT, RPT)], acc_v.at[pl.ds(0, RPT)],
                     sem0).wait()
    pltpu.async_copy(ssrc_hbm.at[pl.ds(t * SCAP, SCAP)], ssrc_v, sem0).wait()
    pltpu.async_copy(gd_hbm.at[pl.ds(t * GCAP, GCAP)], gd_v, sem0).wait()
    pltpu.async_copy(tot_hbm.at[pl.ds(t * 16, 16)], tot_v, sem0).wait()
    nwin = tot_v[pl.ds(0, 16)][0]

    def fix(i, carry):
        s = ssrc_v[pl.ds(i * 16, 16)]
        gdl = plsc.load_gather(gd_v, [(i * 16 + iota) >> 3])
        selfn = jnp.minimum(t * RPT + gdl, NPAD - 1)
        ssrc_v[pl.ds(i * 16, 16)] = jnp.where(s < 0, selfn, s)
        return carry
    lax.fori_loop(0, SCAP // 16, fix, 0)

    def issue(w, par):
        base = w * WROWS
        rbase = par * WROWS
        if par == 0:
            sem = sem0
        else:
            sem = sem1
        pltpu.async_copy(h_hbm.at[ssrc_v.at[pl.ds(base, 96)]],
                         rows_v.at[pl.ds(rbase, 96)], sem)
        pltpu.async_copy(h_hbm.at[ssrc_v.at[pl.ds(base + 96, 96)]],
                         rows_v.at[pl.ds(rbase + 96, 96)], sem)

    def drain(par):
        if par == 0:
            sem = sem0
        else:
            sem = sem1
        rbase = par * WROWS
        pltpu.make_async_copy(h_hbm.at[ssrc_v.at[pl.ds(0, 96)]],
                              rows_v.at[pl.ds(rbase, 96)], sem).wait()
        pltpu.make_async_copy(h_hbm.at[ssrc_v.at[pl.ds(0, 96)]],
                              rows_v.at[pl.ds(rbase + 96, 96)], sem).wait()

    @pl.when(nwin > 0)
    def _():
        issue(0, 0)

    def w_body(w, carry):
        nxt = w + 1

        @pl.when(nxt < nwin)
        def _():
            @pl.when(lax.rem(nxt, 2) == 0)
            def _():
                issue(nxt, 0)

            @pl.when(lax.rem(nxt, 2) == 1)
            def _():
                issue(nxt, 1)

        par = lax.rem(w, 2)

        @pl.when(par == 0)
        def _():
            drain(0)

        @pl.when(par == 1)
        def _():
            drain(1)

        rbase = par * WROWS
        gd0 = gd_v[pl.ds(w * WG, 16)]
        gd1 = gd_v[pl.ds(w * WG + 8, 16)]
        for grp in range(WG):
            if grp < 16:
                d_g = gd0[grp]
            else:
                d_g = gd1[grp - 8]
            rb = rbase + grp * 8
            for j in range(8):
                sl = pl.ds(j * 16, 16)
                m0 = jnp.maximum(rows_v[rb, sl], rows_v[rb + 1, sl])
                m1 = jnp.maximum(rows_v[rb + 2, sl], rows_v[rb + 3, sl])
                m2 = jnp.maximum(rows_v[rb + 4, sl], rows_v[rb + 5, sl])
                m3 = jnp.maximum(rows_v[rb + 6, sl], rows_v[rb + 7, sl])
                m = jnp.maximum(jnp.maximum(m0, m1), jnp.maximum(m2, m3))
                acc_v[d_g, sl] = jnp.maximum(acc_v[d_g, sl], m)
        return carry

    lax.fori_loop(0, nwin, w_body, 0)
    pltpu.async_copy(acc_v.at[pl.ds(0, RPT)], out_hbm.at[pl.ds(t * RPT, RPT)],
                     sem0).wait()


@functools.partial(
    pl.kernel,
    out_type=jax.ShapeDtypeStruct((NPAD, 128), jnp.float32),
    mesh=_mesh(),
    scratch_types=(pltpu.VMEM((RPT + 8, 128), jnp.float32),
                   pltpu.VMEM((SCAP,), jnp.int32),
                   pltpu.VMEM((GCAP,), jnp.int32),
                   pltpu.VMEM((2 * WROWS, 128), jnp.float32),
                   pltpu.VMEM((16,), jnp.int32),
                   pltpu.SemaphoreType.DMA,
                   pltpu.SemaphoreType.DMA),
    compiler_params=_sc_params,
)
def _segmax2(h_hbm, ssrc_hbm, gd_hbm, tot_hbm, out_hbm, *rest):
    _segmax2_body(h_hbm, ssrc_hbm, gd_hbm, tot_hbm, out_hbm, *rest)


def _segmax_body(h_hbm, qsrc_hbm, qdst_hbm, counts_hbm, out_hbm,
                 acc_v, qs_v, qd_v, rows_v, cnt_v, sem0, sem1):
    t = _wid()
    pltpu.async_copy(h_hbm.at[pl.ds(t * RPT, RPT)], acc_v.at[pl.ds(0, RPT)],
                     sem0).wait()
    pltpu.async_copy(counts_hbm, cnt_v, sem0).wait()

    iota = _iota16()

    def w_body(w, carry):
        qoff = (w * NW + t) * CAP
        pltpu.async_copy(qsrc_hbm.at[pl.ds(qoff, CAP)], qs_v, sem0)
        pltpu.async_copy(qdst_hbm.at[pl.ds(qoff, CAP)], qd_v, sem1)
        pltpu.make_async_copy(qsrc_hbm.at[pl.ds(qoff, CAP)], qs_v, sem0).wait()
        pltpu.make_async_copy(qdst_hbm.at[pl.ds(qoff, CAP)], qd_v, sem1).wait()
        cidx = w * NW + t
        crow = cnt_v[pl.ds((cidx // 16) * 16, 16)]
        ncha = jnp.max(jnp.where(iota == cidx - (cidx // 16) * 16, crow, 0))

        @pl.when(ncha > 0)
        def _():
            pltpu.async_copy(h_hbm.at[qs_v.at[pl.ds(0, 16)]],
                             rows_v.at[pl.ds(0, 16)], sem0)

        def g_body(g, carry):
            nxt = g + 1

            @pl.when(nxt < ncha)
            def _():
                @pl.when(lax.rem(nxt, 2) == 0)
                def _():
                    pltpu.async_copy(h_hbm.at[qs_v.at[pl.ds(nxt * 16, 16)]],
                                     rows_v.at[pl.ds(0, 16)], sem0)

                @pl.when(lax.rem(nxt, 2) == 1)
                def _():
                    pltpu.async_copy(h_hbm.at[qs_v.at[pl.ds(nxt * 16, 16)]],
                                     rows_v.at[pl.ds(16, 16)], sem1)

            par = lax.rem(g, 2)

            @pl.when(par == 0)
            def _():
                pltpu.make_async_copy(h_hbm.at[qs_v.at[pl.ds(0, 16)]],
                                      rows_v.at[pl.ds(0, 16)], sem0).wait()

            @pl.when(par == 1)
            def _():
                pltpu.make_async_copy(h_hbm.at[qs_v.at[pl.ds(0, 16)]],
                                      rows_v.at[pl.ds(16, 16)], sem1).wait()

            dvec = qd_v[pl.ds(g * 16, 16)]
            rbase = par * 16
            for e in range(16):
                d_e = dvec[e]
                r = rbase + e
                for j in range(8):
                    sl = pl.ds(j * 16, 16)
                    acc_v[d_e, sl] = jnp.maximum(acc_v[d_e, sl], rows_v[r, sl])
            return carry

        lax.fori_loop(0, ncha, g_body, 0)
        return carry

    lax.fori_loop(0, NW, w_body, 0)
    pltpu.async_copy(acc_v.at[pl.ds(0, RPT)], out_hbm.at[pl.ds(t * RPT, RPT)],
                     sem0).wait()


@functools.partial(
    pl.kernel,
    out_type=jax.ShapeDtypeStruct((NPAD, 128), jnp.float32),
    mesh=_mesh(),
    scratch_types=(pltpu.VMEM((RPT + 8, 128), jnp.float32),
                   pltpu.VMEM((CAP,), jnp.int32),
                   pltpu.VMEM((CAP,), jnp.int32),
                   pltpu.VMEM((32, 128), jnp.float32),
                   pltpu.VMEM((NW * NW,), jnp.int32),
                   pltpu.SemaphoreType.DMA,
                   pltpu.SemaphoreType.DMA),
    compiler_params=_sc_params,
)
def _segmax(h_hbm, qsrc_hbm, qdst_hbm, counts_hbm, out_hbm, *rest):
    _segmax_body(h_hbm, qsrc_hbm, qdst_hbm, counts_hbm, out_hbm, *rest)


def _mm1_kernel(x_ref, w_ref, o_ref):
    o_ref[...] = jnp.dot(x_ref[...], w_ref[...],
                         preferred_element_type=jnp.float32)


def _matmul1(x, w):
    n, k = x.shape
    _, m = w.shape
    grid = pl.cdiv(n, ROW_BLK)
    return pl.pallas_call(
        _mm1_kernel,
        grid=(grid,),
        in_specs=[
            pl.BlockSpec((ROW_BLK, k), lambda i: (i, 0)),
            pl.BlockSpec((k, m), lambda i: (0, 0)),
        ],
        out_specs=pl.BlockSpec((ROW_BLK, m), lambda i: (i, 0)),
        out_shape=jax.ShapeDtypeStruct((n, m), jnp.float32),
    )(x, w)


def _relu_mm_kernel(x_ref, b_ref, w_ref, o_ref):
    h = jnp.maximum(x_ref[...] + b_ref[...], 0.0)
    o_ref[...] = jnp.dot(h, w_ref[...], preferred_element_type=jnp.float32)


def _relu_matmul(x, b, w):
    n, k = x.shape
    _, m = w.shape
    grid = pl.cdiv(n, ROW_BLK)
    return pl.pallas_call(
        _relu_mm_kernel,
        grid=(grid,),
        in_specs=[
            pl.BlockSpec((ROW_BLK, k), lambda i: (i, 0)),
            pl.BlockSpec((1, k), lambda i: (0, 0)),
            pl.BlockSpec((k, m), lambda i: (0, 0)),
        ],
        out_specs=pl.BlockSpec((ROW_BLK, m), lambda i: (i, 0)),
        out_shape=jax.ShapeDtypeStruct((n, m), jnp.float32),
    )(x, b.reshape(1, k), w)


def _tail_kernel(x_ref, h_ref, b_ref, o_ref):
    o_ref[...] = jnp.tanh(x_ref[:, :128] + h_ref[...] + b_ref[...])


def _tail(x, h, b):
    n = N_NODES
    m = 128
    blk = 1000
    return pl.pallas_call(
        _tail_kernel,
        grid=(n // blk,),
        in_specs=[
            pl.BlockSpec((blk, x.shape[1]), lambda i: (i, 0)),
            pl.BlockSpec((blk, m), lambda i: (i, 0)),
            pl.BlockSpec((1, m), lambda i: (0, 0)),
        ],
        out_specs=pl.BlockSpec((blk, m), lambda i: (i, 0)),
        out_shape=jax.ShapeDtypeStruct((n, m), jnp.float32),
    )(x, h, b.reshape(1, m))


def kernel(x, edge_index, W1, b1, W2, b2):
    xp = jnp.pad(x, ((0, NPAD - N_NODES), (0, 0)))
    qsrc, qdst, counts = _bucket_edges(edge_index.reshape(-1))
    ssrc, gd, tot = _merge_sort(qsrc, qdst, counts)
    h1 = _matmul1(xp, W1)
    m1 = _segmax2(h1, ssrc, gd, tot)
    h2 = _relu_matmul(m1, b1, W2)
    m2 = _segmax2(h2, ssrc, gd, tot)
    return _tail(x, m2, b2)


# speedup vs baseline: 7.8121x; 1.2330x over previous
import functools

import jax
import jax.numpy as jnp
from jax import lax
from jax.experimental import pallas as pl
from jax.experimental.pallas import tpu as pltpu
from jax.experimental.pallas import tpu_sc as plsc

N_NODES = 10000
N_EDGES = 320000
NW = 32
RPT = 320
NPAD = NW * RPT
EPW = N_EDGES // NW
CAP = 512
QTOT = NW * NW * CAP
DUMMY = RPT

ROW_BLK = 1024

_mesh = functools.partial(
    plsc.VectorSubcoreMesh, core_axis_name="c", subcore_axis_name="s",
    num_cores=2, num_subcores=16)

_sc_params = pltpu.CompilerParams(needs_layout_passes=False)


def _iota16():
    return lax.iota(jnp.int32, 16)


def _vgather(v, idx):
    return v.at[idx].get(mode="promise_in_bounds")


def _lane(v, e):
    return jnp.max(jnp.where(_iota16() == e, v, 0))


def _wid():
    return lax.axis_index("s") * 2 + lax.axis_index("c")


def _bucket_body(edge_hbm, qsrc_hbm, qdst_hbm, counts_hbm,
                 es_v, ed_v, qs_v, qd_v, cnt_v, ccnt_v, sem):
    w = _wid()
    pltpu.async_copy(edge_hbm.at[pl.ds(w * EPW, EPW)], es_v, sem).wait()
    pltpu.async_copy(edge_hbm.at[pl.ds(N_EDGES + w * EPW, EPW)], ed_v,
                     sem).wait()

    iota = _iota16()
    zeros = jnp.zeros((16,), jnp.int32)
    dummyv = jnp.full((16,), DUMMY, jnp.int32)

    cnt_v[pl.ds(0, 16)] = zeros
    cnt_v[pl.ds(16, 16)] = zeros

    def prefill(r, carry):
        base = r * 256
        for k in range(16):
            qd_v[pl.ds(base + k * 16, 16)] = dummyv
            qs_v[pl.ds(base + k * 16, 16)] = zeros
        return carry
    lax.fori_loop(0, NW * CAP // 256, prefill, 0)

    def body(i, carry):
        s = es_v[pl.ds(i * 16, 16)]
        d = ed_v[pl.ds(i * 16, 16)]
        b = ((d >> 6) * 205) >> 10
        bs, perm = plsc.sort_key_val(b, iota)
        ss = _vgather(s, perm)
        ds = _vgather(d, perm)
        prev = _vgather(bs, jnp.maximum(iota - 1, 0))
        rs = (iota == 0) | (bs != prev)
        sidx = plsc.cummax(jnp.where(rs, iota, 0))
        rank = iota - sidx
        base = plsc.load_gather(cnt_v, [bs])
        pos = base + rank
        nxt = _vgather(rs.astype(jnp.int32), jnp.minimum(iota + 1, 15))
        ls = (iota == 15) | ((iota < 15) & (nxt == 1))
        plsc.store_scatter(qs_v, [bs * CAP + pos], ss)
        plsc.store_scatter(qd_v, [bs * CAP + pos], ds - bs * RPT)
        plsc.store_scatter(cnt_v, [bs], pos + 1, mask=ls)
        return carry
    lax.fori_loop(0, EPW // 16, body, 0)

    ccnt_v[pl.ds(0, 16)] = (cnt_v[pl.ds(0, 16)] + 15) >> 4
    ccnt_v[pl.ds(16, 16)] = (cnt_v[pl.ds(16, 16)] + 15) >> 4

    pltpu.async_copy(qs_v, qsrc_hbm.at[pl.ds(w * NW * CAP, NW * CAP)],
                     sem).wait()
    pltpu.async_copy(qd_v, qdst_hbm.at[pl.ds(w * NW * CAP, NW * CAP)],
                     sem).wait()
    pltpu.async_copy(ccnt_v, counts_hbm.at[pl.ds(w * NW, NW)], sem).wait()


@functools.partial(
    pl.kernel,
    out_type=(jax.ShapeDtypeStruct((QTOT,), jnp.int32),
              jax.ShapeDtypeStruct((QTOT,), jnp.int32),
              jax.ShapeDtypeStruct((NW * NW,), jnp.int32)),
    mesh=_mesh(),
    scratch_types=(pltpu.VMEM((EPW,), jnp.int32),
                   pltpu.VMEM((EPW,), jnp.int32),
                   pltpu.VMEM((NW * CAP,), jnp.int32),
                   pltpu.VMEM((NW * CAP,), jnp.int32),
                   pltpu.VMEM((NW,), jnp.int32),
                   pltpu.VMEM((NW,), jnp.int32),
                   pltpu.SemaphoreType.DMA),
    compiler_params=_sc_params,
)
def _bucket_edges(edge_hbm, qsrc_hbm, qdst_hbm, counts_hbm, *rest):
    _bucket_body(edge_hbm, qsrc_hbm, qdst_hbm, counts_hbm, *rest)


SCAP = 20480
GCAP = 2560
WG = 24
WROWS = WG * 8


def _merge_sort_body(qsrc_hbm, qdst_hbm, counts_hbm,
                     ssrc_hbm, gd_hbm, tot_hbm,
                     qs_v, qd_v, bins_v, off_v, ng_v, cnt2_v,
                     ssrc_v, gd_v, tot_v, sem):
    t = _wid()
    iota = _iota16()
    zeros = jnp.zeros((16,), jnp.int32)

    def stage(w, carry):
        qoff = (w * NW + t) * CAP
        pltpu.async_copy(qsrc_hbm.at[pl.ds(qoff, CAP)],
                         qs_v.at[pl.ds(w * CAP, CAP)], sem)
        pltpu.make_async_copy(qsrc_hbm.at[pl.ds(qoff, CAP)],
                              qs_v.at[pl.ds(w * CAP, CAP)], sem).wait()
        pltpu.async_copy(qdst_hbm.at[pl.ds(qoff, CAP)],
                         qd_v.at[pl.ds(w * CAP, CAP)], sem)
        pltpu.make_async_copy(qdst_hbm.at[pl.ds(qoff, CAP)],
                              qd_v.at[pl.ds(w * CAP, CAP)], sem).wait()
        return carry
    lax.fori_loop(0, NW, stage, 0)

    for k in range(328 // 8 // 2):
        bins_v[pl.ds(k * 16, 16)] = zeros
        cnt2_v[pl.ds(k * 16, 16)] = zeros
    bins_v[pl.ds(328 - 16, 16)] = zeros
    cnt2_v[pl.ds(328 - 16, 16)] = zeros

    neg1 = jnp.full((16,), -1, jnp.int32)
    dum = jnp.full((16,), DUMMY, jnp.int32)

    def pre1(i, carry):
        ssrc_v[pl.ds(i * 16, 16)] = neg1
        return carry
    lax.fori_loop(0, SCAP // 16, pre1, 0)

    def pre2(i, carry):
        gd_v[pl.ds(i * 16, 16)] = dum
        return carry
    lax.fori_loop(0, GCAP // 16, pre2, 0)

    def hist(i, carry):
        d = qd_v[pl.ds(i * 16, 16)]
        ds, _ = plsc.sort_key_val(d, iota)
        prev = _vgather(ds, jnp.maximum(iota - 1, 0))
        rs = (iota == 0) | (ds != prev)
        sidx = plsc.cummax(jnp.where(rs, iota, 0))
        rank = iota - sidx
        nxt = _vgather(rs.astype(jnp.int32), jnp.minimum(iota + 1, 15))
        ls = (iota == 15) | ((iota < 15) & (nxt == 1))
        base = plsc.load_gather(bins_v, [ds])
        plsc.store_scatter(bins_v, [ds], base + rank + 1, mask=ls)
        return carry
    lax.fori_loop(0, NW * CAP // 16, hist, 0)

    def prefix(v, carry):
        b = bins_v[pl.ds(v * 16, 16)]
        pad8 = (b + 7) & ~7
        cs = plsc.cumsum(pad8)
        off_v[pl.ds(v * 16, 16)] = carry + cs - pad8
        ng_v[pl.ds(v * 16, 16)] = pad8 >> 3
        return carry + cs[15]
    carry = lax.fori_loop(0, 320 // 16, prefix, 0)

    def gdfill(v, carry):
        offv = off_v[pl.ds(v * 16, 16)]
        ngv = ng_v[pl.ds(v * 16, 16)]
        for lane in range(16):
            goff = offv[lane] >> 3
            n = ngv[lane]
            dval = jnp.full((16,), v * 16 + lane, jnp.int32)

            def put(k, c):
                plsc.store_scatter(gd_v, [jnp.where(iota == 0, goff + k, 0)],
                                   dval, mask=(iota == 0))
                return c
            lax.fori_loop(0, n, put, 0)
        return carry
    lax.fori_loop(0, 20, gdfill, 0)

    def place(i, carry):
        s = qs_v[pl.ds(i * 16, 16)]
        d = qd_v[pl.ds(i * 16, 16)]
        ds, perm = plsc.sort_key_val(d, iota)
        ss = _vgather(s, perm)
        prev = _vgather(ds, jnp.maximum(iota - 1, 0))
        rs = (iota == 0) | (ds != prev)
        sidx = plsc.cummax(jnp.where(rs, iota, 0))
        rank = iota - sidx
        nxt = _vgather(rs.astype(jnp.int32), jnp.minimum(iota + 1, 15))
        ls = (iota == 15) | ((iota < 15) & (nxt == 1))
        valid = ds != DUMMY
        base = plsc.load_gather(cnt2_v, [ds])
        tgt = plsc.load_gather(off_v, [jnp.minimum(ds, 319)])
        plsc.store_scatter(ssrc_v, [jnp.minimum(tgt + base + rank, SCAP - 1)],
                           ss, mask=valid)
        plsc.store_scatter(cnt2_v, [ds], base + rank + 1, mask=ls)
        return carry
    lax.fori_loop(0, NW * CAP // 16, place, 0)

    g_tot = carry >> 3
    nwin = ((g_tot + WG - 1) * 2731) >> 16
    tot_v[pl.ds(0, 16)] = jnp.where(iota == 0, nwin, 0)

    pltpu.async_copy(ssrc_v, ssrc_hbm.at[pl.ds(t * SCAP, SCAP)], sem).wait()
    pltpu.async_copy(gd_v, gd_hbm.at[pl.ds(t * GCAP, GCAP)], sem).wait()
    pltpu.async_copy(tot_v, tot_hbm.at[pl.ds(t * 16, 16)], sem).wait()


@functools.partial(
    pl.kernel,
    out_type=(jax.ShapeDtypeStruct((NW * SCAP,), jnp.int32),
              jax.ShapeDtypeStruct((NW * GCAP,), jnp.int32),
              jax.ShapeDtypeStruct((NW * 16,), jnp.int32)),
    mesh=_mesh(),
    scratch_types=(pltpu.VMEM((NW * CAP,), jnp.int32),
                   pltpu.VMEM((NW * CAP,), jnp.int32),
                   pltpu.VMEM((328,), jnp.int32),
                   pltpu.VMEM((328,), jnp.int32),
                   pltpu.VMEM((320,), jnp.int32),
                   pltpu.VMEM((328,), jnp.int32),
                   pltpu.VMEM((SCAP,), jnp.int32),
                   pltpu.VMEM((GCAP,), jnp.int32),
                   pltpu.VMEM((16,), jnp.int32),
                   pltpu.SemaphoreType.DMA),
    compiler_params=_sc_params,
)
def _merge_sort(qsrc_hbm, qdst_hbm, counts_hbm, ssrc_hbm, gd_hbm, tot_hbm,
                *rest):
    _merge_sort_body(qsrc_hbm, qdst_hbm, counts_hbm, ssrc_hbm, gd_hbm,
                     tot_hbm, *rest)


def _segmax2_body(h_hbm, ssrc_hbm, gd_hbm, tot_hbm, out_hbm,
                  acc_v, ssrc_v, gd_v, rows_v, tot_v, sem0, sem1):
    t = _wid()
    iota = _iota16()
    pltpu.async_copy(h_hbm.at[pl.ds(t * RPT, RPT)], acc_v.at[pl.ds(0, RPT)],
                     sem0).wait()
    pltpu.async_copy(ssrc_hbm.at[pl.ds(t * SCAP, SCAP)], ssrc_v, sem0).wait()
    pltpu.async_copy(gd_hbm.at[pl.ds(t * GCAP, GCAP)], gd_v, sem0).wait()
    pltpu.async_copy(tot_hbm.at[pl.ds(t * 16, 16)], tot_v, sem0).wait()
    nwin = tot_v[pl.ds(0, 16)][0]

    def fix(i, carry):
        s = ssrc_v[pl.ds(i * 16, 16)]
        gdl = plsc.load_gather(gd_v, [(i * 16 + iota) >> 3])
        selfn = jnp.minimum(t * RPT + gdl, NPAD - 1)
        ssrc_v[pl.ds(i * 16, 16)] = jnp.where(s < 0, selfn, s)
        return carry
    lax.fori_loop(0, SCAP // 16, fix, 0)

    def issue(w, par):
        base = w * WROWS
        rbase = par * WROWS
        if par == 0:
            sem = sem0
        else:
            sem = sem1
        pltpu.async_copy(h_hbm.at[ssrc_v.at[pl.ds(base, 96)]],
                         rows_v.at[pl.ds(rbase, 96)], sem)
        pltpu.async_copy(h_hbm.at[ssrc_v.at[pl.ds(base + 96, 96)]],
                         rows_v.at[pl.ds(rbase + 96, 96)], sem)

    def drain(par):
        if par == 0:
            sem = sem0
        else:
            sem = sem1
        rbase = par * WROWS
        pltpu.make_async_copy(h_hbm.at[ssrc_v.at[pl.ds(0, 96)]],
                              rows_v.at[pl.ds(rbase, 96)], sem).wait()
        pltpu.make_async_copy(h_hbm.at[ssrc_v.at[pl.ds(0, 96)]],
                              rows_v.at[pl.ds(rbase + 96, 96)], sem).wait()

    @pl.when(nwin > 0)
    def _():
        issue(0, 0)

    def w_body(w, carry):
        nxt = w + 1

        @pl.when(nxt < nwin)
        def _():
            @pl.when(lax.rem(nxt, 2) == 0)
            def _():
                issue(nxt, 0)

            @pl.when(lax.rem(nxt, 2) == 1)
            def _():
                issue(nxt, 1)

        par = lax.rem(w, 2)
        gd0 = gd_v[pl.ds(w * WG, 16)]
        gd1 = gd_v[pl.ds(w * WG + 8, 16)]

        def process(rbase):
            for grp in range(WG):
                if grp < 16:
                    d_g = gd0[grp]
                else:
                    d_g = gd1[grp - 8]
                rb = rbase + grp * 8
                ms = []
                for j in range(8):
                    sl = pl.ds(j * 16, 16)
                    m0 = jnp.maximum(rows_v[rb, sl], rows_v[rb + 1, sl])
                    m1 = jnp.maximum(rows_v[rb + 2, sl], rows_v[rb + 3, sl])
                    m2 = jnp.maximum(rows_v[rb + 4, sl], rows_v[rb + 5, sl])
                    m3 = jnp.maximum(rows_v[rb + 6, sl], rows_v[rb + 7, sl])
                    ms.append(jnp.maximum(jnp.maximum(m0, m1),
                                          jnp.maximum(m2, m3)))
                for j in range(8):
                    sl = pl.ds(j * 16, 16)
                    acc_v[d_g, sl] = jnp.maximum(acc_v[d_g, sl], ms[j])

        @pl.when(par == 0)
        def _():
            drain(0)
            process(0)

        @pl.when(par == 1)
        def _():
            drain(1)
            process(WROWS)
        return carry

    lax.fori_loop(0, nwin, w_body, 0)
    pltpu.async_copy(acc_v.at[pl.ds(0, RPT)], out_hbm.at[pl.ds(t * RPT, RPT)],
                     sem0).wait()


@functools.partial(
    pl.kernel,
    out_type=jax.ShapeDtypeStruct((NPAD, 128), jnp.float32),
    mesh=_mesh(),
    scratch_types=(pltpu.VMEM((RPT + 8, 128), jnp.float32),
                   pltpu.VMEM((SCAP,), jnp.int32),
                   pltpu.VMEM((GCAP,), jnp.int32),
                   pltpu.VMEM((2 * WROWS, 128), jnp.float32),
                   pltpu.VMEM((16,), jnp.int32),
                   pltpu.SemaphoreType.DMA,
                   pltpu.SemaphoreType.DMA),
    compiler_params=_sc_params,
)
def _segmax2(h_hbm, ssrc_hbm, gd_hbm, tot_hbm, out_hbm, *rest):
    _segmax2_body(h_hbm, ssrc_hbm, gd_hbm, tot_hbm, out_hbm, *rest)


def _segmax_body(h_hbm, qsrc_hbm, qdst_hbm, counts_hbm, out_hbm,
                 acc_v, qs_v, qd_v, rows_v, cnt_v, sem0, sem1):
    t = _wid()
    pltpu.async_copy(h_hbm.at[pl.ds(t * RPT, RPT)], acc_v.at[pl.ds(0, RPT)],
                     sem0).wait()
    pltpu.async_copy(counts_hbm, cnt_v, sem0).wait()

    iota = _iota16()

    def w_body(w, carry):
        qoff = (w * NW + t) * CAP
        pltpu.async_copy(qsrc_hbm.at[pl.ds(qoff, CAP)], qs_v, sem0)
        pltpu.async_copy(qdst_hbm.at[pl.ds(qoff, CAP)], qd_v, sem1)
        pltpu.make_async_copy(qsrc_hbm.at[pl.ds(qoff, CAP)], qs_v, sem0).wait()
        pltpu.make_async_copy(qdst_hbm.at[pl.ds(qoff, CAP)], qd_v, sem1).wait()
        cidx = w * NW + t
        crow = cnt_v[pl.ds((cidx // 16) * 16, 16)]
        ncha = jnp.max(jnp.where(iota == cidx - (cidx // 16) * 16, crow, 0))

        @pl.when(ncha > 0)
        def _():
            pltpu.async_copy(h_hbm.at[qs_v.at[pl.ds(0, 16)]],
                             rows_v.at[pl.ds(0, 16)], sem0)

        def g_body(g, carry):
            nxt = g + 1

            @pl.when(nxt < ncha)
            def _():
                @pl.when(lax.rem(nxt, 2) == 0)
                def _():
                    pltpu.async_copy(h_hbm.at[qs_v.at[pl.ds(nxt * 16, 16)]],
                                     rows_v.at[pl.ds(0, 16)], sem0)

                @pl.when(lax.rem(nxt, 2) == 1)
                def _():
                    pltpu.async_copy(h_hbm.at[qs_v.at[pl.ds(nxt * 16, 16)]],
                                     rows_v.at[pl.ds(16, 16)], sem1)

            par = lax.rem(g, 2)

            @pl.when(par == 0)
            def _():
                pltpu.make_async_copy(h_hbm.at[qs_v.at[pl.ds(0, 16)]],
                                      rows_v.at[pl.ds(0, 16)], sem0).wait()

            @pl.when(par == 1)
            def _():
                pltpu.make_async_copy(h_hbm.at[qs_v.at[pl.ds(0, 16)]],
                                      rows_v.at[pl.ds(16, 16)], sem1).wait()

            dvec = qd_v[pl.ds(g * 16, 16)]
            rbase = par * 16
            for e in range(16):
                d_e = dvec[e]
                r = rbase + e
                for j in range(8):
                    sl = pl.ds(j * 16, 16)
                    acc_v[d_e, sl] = jnp.maximum(acc_v[d_e, sl], rows_v[r, sl])
            return carry

        lax.fori_loop(0, ncha, g_body, 0)
        return carry

    lax.fori_loop(0, NW, w_body, 0)
    pltpu.async_copy(acc_v.at[pl.ds(0, RPT)], out_hbm.at[pl.ds(t * RPT, RPT)],
                     sem0).wait()


@functools.partial(
    pl.kernel,
    out_type=jax.ShapeDtypeStruct((NPAD, 128), jnp.float32),
    mesh=_mesh(),
    scratch_types=(pltpu.VMEM((RPT + 8, 128), jnp.float32),
                   pltpu.VMEM((CAP,), jnp.int32),
                   pltpu.VMEM((CAP,), jnp.int32),
                   pltpu.VMEM((32, 128), jnp.float32),
                   pltpu.VMEM((NW * NW,), jnp.int32),
                   pltpu.SemaphoreType.DMA,
                   pltpu.SemaphoreType.DMA),
    compiler_params=_sc_params,
)
def _segmax(h_hbm, qsrc_hbm, qdst_hbm, counts_hbm, out_hbm, *rest):
    _segmax_body(h_hbm, qsrc_hbm, qdst_hbm, counts_hbm, out_hbm, *rest)


def _mm1_kernel(x_ref, w_ref, o_ref):
    o_ref[...] = jnp.dot(x_ref[...], w_ref[...],
                         preferred_element_type=jnp.float32)


def _matmul1(x, w):
    n, k = x.shape
    _, m = w.shape
    grid = pl.cdiv(n, ROW_BLK)
    return pl.pallas_call(
        _mm1_kernel,
        grid=(grid,),
        in_specs=[
            pl.BlockSpec((ROW_BLK, k), lambda i: (i, 0)),
            pl.BlockSpec((k, m), lambda i: (0, 0)),
        ],
        out_specs=pl.BlockSpec((ROW_BLK, m), lambda i: (i, 0)),
        out_shape=jax.ShapeDtypeStruct((n, m), jnp.float32),
    )(x, w)


def _relu_mm_kernel(x_ref, b_ref, w_ref, o_ref):
    h = jnp.maximum(x_ref[...] + b_ref[...], 0.0)
    o_ref[...] = jnp.dot(h, w_ref[...], preferred_element_type=jnp.float32)


def _relu_matmul(x, b, w):
    n, k = x.shape
    _, m = w.shape
    grid = pl.cdiv(n, ROW_BLK)
    return pl.pallas_call(
        _relu_mm_kernel,
        grid=(grid,),
        in_specs=[
            pl.BlockSpec((ROW_BLK, k), lambda i: (i, 0)),
            pl.BlockSpec((1, k), lambda i: (0, 0)),
            pl.BlockSpec((k, m), lambda i: (0, 0)),
        ],
        out_specs=pl.BlockSpec((ROW_BLK, m), lambda i: (i, 0)),
        out_shape=jax.ShapeDtypeStruct((n, m), jnp.float32),
    )(x, b.reshape(1, k), w)


def _tail_kernel(x_ref, h_ref, b_ref, o_ref):
    o_ref[...] = jnp.tanh(x_ref[:, :128] + h_ref[...] + b_ref[...])


def _tail(x, h, b):
    n = N_NODES
    m = 128
    blk = 1000
    return pl.pallas_call(
        _tail_kernel,
        grid=(n // blk,),
        in_specs=[
            pl.BlockSpec((blk, x.shape[1]), lambda i: (i, 0)),
            pl.BlockSpec((blk, m), lambda i: (i, 0)),
            pl.BlockSpec((1, m), lambda i: (0, 0)),
        ],
        out_specs=pl.BlockSpec((blk, m), lambda i: (i, 0)),
        out_shape=jax.ShapeDtypeStruct((n, m), jnp.float32),
    )(x, h, b.reshape(1, m))


def kernel(x, edge_index, W1, b1, W2, b2):
    xp = jnp.pad(x, ((0, NPAD - N_NODES), (0, 0)))
    qsrc, qdst, counts = _bucket_edges(edge_index.reshape(-1))
    ssrc, gd, tot = _merge_sort(qsrc, qdst, counts)
    h1 = _matmul1(xp, W1)
    m1 = _segmax2(h1, ssrc, gd, tot)
    h2 = _relu_matmul(m1, b1, W2)
    m2 = _segmax2(h2, ssrc, gd, tot)
    return _tail(x, m2, b2)


# speedup vs baseline: 8.4037x; 1.0757x over previous
import functools

import jax
import jax.numpy as jnp
from jax import lax
from jax.experimental import pallas as pl
from jax.experimental.pallas import tpu as pltpu
from jax.experimental.pallas import tpu_sc as plsc

N_NODES = 10000
N_EDGES = 320000
NW = 32
RPT = 320
NPAD = NW * RPT
EPW = N_EDGES // NW
CAP = 512
QTOT = NW * NW * CAP
DUMMY = RPT

ROW_BLK = 1024

_mesh = functools.partial(
    plsc.VectorSubcoreMesh, core_axis_name="c", subcore_axis_name="s",
    num_cores=2, num_subcores=16)

_sc_params = pltpu.CompilerParams(needs_layout_passes=False)


def _iota16():
    return lax.iota(jnp.int32, 16)


def _vgather(v, idx):
    return v.at[idx].get(mode="promise_in_bounds")


def _lane(v, e):
    return jnp.max(jnp.where(_iota16() == e, v, 0))


def _wid():
    return lax.axis_index("s") * 2 + lax.axis_index("c")


def _bucket_body(edge_hbm, qsrc_hbm, qdst_hbm, counts_hbm,
                 es_v, ed_v, qs_v, qd_v, cnt_v, ccnt_v, sem):
    w = _wid()
    pltpu.async_copy(edge_hbm.at[pl.ds(w * EPW, EPW)], es_v, sem).wait()
    pltpu.async_copy(edge_hbm.at[pl.ds(N_EDGES + w * EPW, EPW)], ed_v,
                     sem).wait()

    iota = _iota16()
    zeros = jnp.zeros((16,), jnp.int32)
    dummyv = jnp.full((16,), DUMMY, jnp.int32)

    cnt_v[pl.ds(0, 16)] = zeros
    cnt_v[pl.ds(16, 16)] = zeros

    def prefill(r, carry):
        base = r * 256
        for k in range(16):
            qd_v[pl.ds(base + k * 16, 16)] = dummyv
            qs_v[pl.ds(base + k * 16, 16)] = zeros
        return carry
    lax.fori_loop(0, NW * CAP // 256, prefill, 0)

    def body(i, carry):
        s = es_v[pl.ds(i * 16, 16)]
        d = ed_v[pl.ds(i * 16, 16)]
        b = ((d >> 6) * 205) >> 10
        bs, perm = plsc.sort_key_val(b, iota)
        ss = _vgather(s, perm)
        ds = _vgather(d, perm)
        prev = _vgather(bs, jnp.maximum(iota - 1, 0))
        rs = (iota == 0) | (bs != prev)
        sidx = plsc.cummax(jnp.where(rs, iota, 0))
        rank = iota - sidx
        base = plsc.load_gather(cnt_v, [bs])
        pos = base + rank
        nxt = _vgather(rs.astype(jnp.int32), jnp.minimum(iota + 1, 15))
        ls = (iota == 15) | ((iota < 15) & (nxt == 1))
        plsc.store_scatter(qs_v, [bs * CAP + pos], ss)
        plsc.store_scatter(qd_v, [bs * CAP + pos], ds - bs * RPT)
        plsc.store_scatter(cnt_v, [bs], pos + 1, mask=ls)
        return carry
    lax.fori_loop(0, EPW // 16, body, 0)

    ccnt_v[pl.ds(0, 16)] = (cnt_v[pl.ds(0, 16)] + 15) >> 4
    ccnt_v[pl.ds(16, 16)] = (cnt_v[pl.ds(16, 16)] + 15) >> 4

    pltpu.async_copy(qs_v, qsrc_hbm.at[pl.ds(w * NW * CAP, NW * CAP)],
                     sem).wait()
    pltpu.async_copy(qd_v, qdst_hbm.at[pl.ds(w * NW * CAP, NW * CAP)],
                     sem).wait()
    pltpu.async_copy(ccnt_v, counts_hbm.at[pl.ds(w * NW, NW)], sem).wait()


@functools.partial(
    pl.kernel,
    out_type=(jax.ShapeDtypeStruct((QTOT,), jnp.int32),
              jax.ShapeDtypeStruct((QTOT,), jnp.int32),
              jax.ShapeDtypeStruct((NW * NW,), jnp.int32)),
    mesh=_mesh(),
    scratch_types=(pltpu.VMEM((EPW,), jnp.int32),
                   pltpu.VMEM((EPW,), jnp.int32),
                   pltpu.VMEM((NW * CAP,), jnp.int32),
                   pltpu.VMEM((NW * CAP,), jnp.int32),
                   pltpu.VMEM((NW,), jnp.int32),
                   pltpu.VMEM((NW,), jnp.int32),
                   pltpu.SemaphoreType.DMA),
    compiler_params=_sc_params,
)
def _bucket_edges(edge_hbm, qsrc_hbm, qdst_hbm, counts_hbm, *rest):
    _bucket_body(edge_hbm, qsrc_hbm, qdst_hbm, counts_hbm, *rest)


SCAP = 20480
GCAP = 2560
WG = 24
WROWS = WG * 8


def _merge_sort_body(qsrc_hbm, qdst_hbm, counts_hbm,
                     ssrc_hbm, gd_hbm, tot_hbm,
                     qs_v, qd_v, bins_v, off_v, ng_v, cnt2_v,
                     ssrc_v, gd_v, tot_v, cntk_v, sem):
    t = _wid()
    iota = _iota16()
    zeros = jnp.zeros((16,), jnp.int32)

    def stage_issue(w, carry):
        qoff = (w * NW + t) * CAP
        pltpu.async_copy(qsrc_hbm.at[pl.ds(qoff, CAP)],
                         qs_v.at[pl.ds(w * CAP, CAP)], sem)
        pltpu.async_copy(qdst_hbm.at[pl.ds(qoff, CAP)],
                         qd_v.at[pl.ds(w * CAP, CAP)], sem)
        return carry
    lax.fori_loop(0, NW, stage_issue, 0)
    pltpu.async_copy(counts_hbm, cntk_v, sem)

    def stage_drain(w, carry):
        qoff = (w * NW + t) * CAP
        pltpu.make_async_copy(qsrc_hbm.at[pl.ds(qoff, CAP)],
                              qs_v.at[pl.ds(w * CAP, CAP)], sem).wait()
        pltpu.make_async_copy(qdst_hbm.at[pl.ds(qoff, CAP)],
                              qd_v.at[pl.ds(w * CAP, CAP)], sem).wait()
        return carry
    lax.fori_loop(0, NW, stage_drain, 0)
    pltpu.make_async_copy(counts_hbm, cntk_v, sem).wait()

    def _nch(w):
        row = cntk_v[pl.ds(w * NW + (t // 16) * 16, 16)]
        return jnp.max(jnp.where(iota == t - (t // 16) * 16, row, 0))

    for k in range(328 // 8 // 2):
        bins_v[pl.ds(k * 16, 16)] = zeros
        cnt2_v[pl.ds(k * 16, 16)] = zeros
    bins_v[pl.ds(328 - 16, 16)] = zeros
    cnt2_v[pl.ds(328 - 16, 16)] = zeros

    neg1 = jnp.full((16,), -1, jnp.int32)
    dum = jnp.full((16,), DUMMY, jnp.int32)

    def pre1(i, carry):
        ssrc_v[pl.ds(i * 16, 16)] = neg1
        return carry
    lax.fori_loop(0, SCAP // 16, pre1, 0)

    def pre2(i, carry):
        gd_v[pl.ds(i * 16, 16)] = dum
        return carry
    lax.fori_loop(0, GCAP // 16, pre2, 0)

    def hist_w(w, carry):
        qbase = w * CAP

        def hist(j, c):
            d = qd_v[pl.ds(qbase + j * 16, 16)]
            ds, _ = plsc.sort_key_val(d, iota)
            prev = _vgather(ds, jnp.maximum(iota - 1, 0))
            rs = (iota == 0) | (ds != prev)
            sidx = plsc.cummax(jnp.where(rs, iota, 0))
            rank = iota - sidx
            nxt = _vgather(rs.astype(jnp.int32), jnp.minimum(iota + 1, 15))
            ls = (iota == 15) | ((iota < 15) & (nxt == 1))
            base = plsc.load_gather(bins_v, [ds])
            plsc.store_scatter(bins_v, [ds], base + rank + 1, mask=ls)
            return c
        lax.fori_loop(0, _nch(w), hist, 0)
        return carry
    lax.fori_loop(0, NW, hist_w, 0)

    def prefix(v, carry):
        b = bins_v[pl.ds(v * 16, 16)]
        pad8 = (b + 7) & ~7
        cs = plsc.cumsum(pad8)
        off_v[pl.ds(v * 16, 16)] = carry + cs - pad8
        ng_v[pl.ds(v * 16, 16)] = pad8 >> 3
        return carry + cs[15]
    carry = lax.fori_loop(0, 320 // 16, prefix, 0)

    def gdfill(v, carry):
        offv = off_v[pl.ds(v * 16, 16)]
        ngv = ng_v[pl.ds(v * 16, 16)]
        for lane in range(16):
            goff = offv[lane] >> 3
            n = ngv[lane]
            dval = jnp.full((16,), v * 16 + lane, jnp.int32)

            def put(k, c):
                plsc.store_scatter(gd_v, [jnp.where(iota == 0, goff + k, 0)],
                                   dval, mask=(iota == 0))
                return c
            lax.fori_loop(0, n, put, 0)
        return carry
    lax.fori_loop(0, 20, gdfill, 0)

    def place_w(w, carry):
        qbase = w * CAP

        def place(j, c):
            s = qs_v[pl.ds(qbase + j * 16, 16)]
            d = qd_v[pl.ds(qbase + j * 16, 16)]
            ds, perm = plsc.sort_key_val(d, iota)
            ss = _vgather(s, perm)
            prev = _vgather(ds, jnp.maximum(iota - 1, 0))
            rs = (iota == 0) | (ds != prev)
            sidx = plsc.cummax(jnp.where(rs, iota, 0))
            rank = iota - sidx
            nxt = _vgather(rs.astype(jnp.int32), jnp.minimum(iota + 1, 15))
            ls = (iota == 15) | ((iota < 15) & (nxt == 1))
            valid = ds != DUMMY
            base = plsc.load_gather(cnt2_v, [ds])
            tgt = plsc.load_gather(off_v, [jnp.minimum(ds, 319)])
            plsc.store_scatter(ssrc_v,
                               [jnp.minimum(tgt + base + rank, SCAP - 1)],
                               ss, mask=valid)
            plsc.store_scatter(cnt2_v, [ds], base + rank + 1, mask=ls)
            return c
        lax.fori_loop(0, _nch(w), place, 0)
        return carry
    lax.fori_loop(0, NW, place_w, 0)

    g_tot = carry >> 3
    nwin = ((g_tot + WG - 1) * 2731) >> 16
    tot_v[pl.ds(0, 16)] = jnp.where(iota == 0, nwin, 0)

    pltpu.async_copy(ssrc_v, ssrc_hbm.at[pl.ds(t * SCAP, SCAP)], sem).wait()
    pltpu.async_copy(gd_v, gd_hbm.at[pl.ds(t * GCAP, GCAP)], sem).wait()
    pltpu.async_copy(tot_v, tot_hbm.at[pl.ds(t * 16, 16)], sem).wait()


@functools.partial(
    pl.kernel,
    out_type=(jax.ShapeDtypeStruct((NW * SCAP,), jnp.int32),
              jax.ShapeDtypeStruct((NW * GCAP,), jnp.int32),
              jax.ShapeDtypeStruct((NW * 16,), jnp.int32)),
    mesh=_mesh(),
    scratch_types=(pltpu.VMEM((NW * CAP,), jnp.int32),
                   pltpu.VMEM((NW * CAP,), jnp.int32),
                   pltpu.VMEM((328,), jnp.int32),
                   pltpu.VMEM((328,), jnp.int32),
                   pltpu.VMEM((320,), jnp.int32),
                   pltpu.VMEM((328,), jnp.int32),
                   pltpu.VMEM((SCAP,), jnp.int32),
                   pltpu.VMEM((GCAP,), jnp.int32),
                   pltpu.VMEM((16,), jnp.int32),
                   pltpu.VMEM((NW * NW,), jnp.int32),
                   pltpu.SemaphoreType.DMA),
    compiler_params=_sc_params,
)
def _merge_sort(qsrc_hbm, qdst_hbm, counts_hbm, ssrc_hbm, gd_hbm, tot_hbm,
                *rest):
    _merge_sort_body(qsrc_hbm, qdst_hbm, counts_hbm, ssrc_hbm, gd_hbm,
                     tot_hbm, *rest)


def _segmax2_body(h_hbm, ssrc_hbm, gd_hbm, tot_hbm, out_hbm,
                  acc_v, ssrc_v, gd_v, rows_v, tot_v, sem0, sem1):
    t = _wid()
    iota = _iota16()
    pltpu.async_copy(h_hbm.at[pl.ds(t * RPT, RPT)], acc_v.at[pl.ds(0, RPT)],
                     sem0)
    pltpu.async_copy(ssrc_hbm.at[pl.ds(t * SCAP, SCAP)], ssrc_v, sem0)
    pltpu.async_copy(gd_hbm.at[pl.ds(t * GCAP, GCAP)], gd_v, sem0)
    pltpu.async_copy(tot_hbm.at[pl.ds(t * 16, 16)], tot_v, sem0)
    pltpu.make_async_copy(h_hbm.at[pl.ds(t * RPT, RPT)],
                          acc_v.at[pl.ds(0, RPT)], sem0).wait()
    pltpu.make_async_copy(ssrc_hbm.at[pl.ds(t * SCAP, SCAP)], ssrc_v,
                          sem0).wait()
    pltpu.make_async_copy(gd_hbm.at[pl.ds(t * GCAP, GCAP)], gd_v,
                          sem0).wait()
    pltpu.make_async_copy(tot_hbm.at[pl.ds(t * 16, 16)], tot_v, sem0).wait()
    nwin = tot_v[pl.ds(0, 16)][0]

    def fix(i, carry):
        s = ssrc_v[pl.ds(i * 16, 16)]
        gdl = plsc.load_gather(gd_v, [(i * 16 + iota) >> 3])
        selfn = jnp.minimum(t * RPT + gdl, NPAD - 1)
        ssrc_v[pl.ds(i * 16, 16)] = jnp.where(s < 0, selfn, s)
        return carry
    lax.fori_loop(0, SCAP // 16, fix, 0)

    def issue(w, par):
        base = w * WROWS
        rbase = par * WROWS
        if par == 0:
            sem = sem0
        else:
            sem = sem1
        pltpu.async_copy(h_hbm.at[ssrc_v.at[pl.ds(base, 96)]],
                         rows_v.at[pl.ds(rbase, 96)], sem)
        pltpu.async_copy(h_hbm.at[ssrc_v.at[pl.ds(base + 96, 96)]],
                         rows_v.at[pl.ds(rbase + 96, 96)], sem)

    def drain(par):
        if par == 0:
            sem = sem0
        else:
            sem = sem1
        rbase = par * WROWS
        pltpu.make_async_copy(h_hbm.at[ssrc_v.at[pl.ds(0, 96)]],
                              rows_v.at[pl.ds(rbase, 96)], sem).wait()
        pltpu.make_async_copy(h_hbm.at[ssrc_v.at[pl.ds(0, 96)]],
                              rows_v.at[pl.ds(rbase + 96, 96)], sem).wait()

    @pl.when(nwin > 0)
    def _():
        issue(0, 0)

    def w_body(w, carry):
        nxt = w + 1

        @pl.when(nxt < nwin)
        def _():
            @pl.when(lax.rem(nxt, 2) == 0)
            def _():
                issue(nxt, 0)

            @pl.when(lax.rem(nxt, 2) == 1)
            def _():
                issue(nxt, 1)

        par = lax.rem(w, 2)
        gd0 = gd_v[pl.ds(w * WG, 16)]
        gd1 = gd_v[pl.ds(w * WG + 8, 16)]

        def process(rbase):
            for grp in range(WG):
                if grp < 16:
                    d_g = gd0[grp]
                else:
                    d_g = gd1[grp - 8]
                rb = rbase + grp * 8
                ms = []
                for j in range(8):
                    sl = pl.ds(j * 16, 16)
                    m0 = jnp.maximum(rows_v[rb, sl], rows_v[rb + 1, sl])
                    m1 = jnp.maximum(rows_v[rb + 2, sl], rows_v[rb + 3, sl])
                    m2 = jnp.maximum(rows_v[rb + 4, sl], rows_v[rb + 5, sl])
                    m3 = jnp.maximum(rows_v[rb + 6, sl], rows_v[rb + 7, sl])
                    ms.append(jnp.maximum(jnp.maximum(m0, m1),
                                          jnp.maximum(m2, m3)))
                for j in range(8):
                    sl = pl.ds(j * 16, 16)
                    acc_v[d_g, sl] = jnp.maximum(acc_v[d_g, sl], ms[j])

        @pl.when(par == 0)
        def _():
            drain(0)
            process(0)

        @pl.when(par == 1)
        def _():
            drain(1)
            process(WROWS)
        return carry

    lax.fori_loop(0, nwin, w_body, 0)
    pltpu.async_copy(acc_v.at[pl.ds(0, RPT)], out_hbm.at[pl.ds(t * RPT, RPT)],
                     sem0).wait()


@functools.partial(
    pl.kernel,
    out_type=jax.ShapeDtypeStruct((NPAD, 128), jnp.float32),
    mesh=_mesh(),
    scratch_types=(pltpu.VMEM((RPT + 8, 128), jnp.float32),
                   pltpu.VMEM((SCAP,), jnp.int32),
                   pltpu.VMEM((GCAP,), jnp.int32),
                   pltpu.VMEM((2 * WROWS, 128), jnp.float32),
                   pltpu.VMEM((16,), jnp.int32),
                   pltpu.SemaphoreType.DMA,
                   pltpu.SemaphoreType.DMA),
    compiler_params=_sc_params,
)
def _segmax2(h_hbm, ssrc_hbm, gd_hbm, tot_hbm, out_hbm, *rest):
    _segmax2_body(h_hbm, ssrc_hbm, gd_hbm, tot_hbm, out_hbm, *rest)


def _segmax_body(h_hbm, qsrc_hbm, qdst_hbm, counts_hbm, out_hbm,
                 acc_v, qs_v, qd_v, rows_v, cnt_v, sem0, sem1):
    t = _wid()
    pltpu.async_copy(h_hbm.at[pl.ds(t * RPT, RPT)], acc_v.at[pl.ds(0, RPT)],
                     sem0).wait()
    pltpu.async_copy(counts_hbm, cnt_v, sem0).wait()

    iota = _iota16()

    def w_body(w, carry):
        qoff = (w * NW + t) * CAP
        pltpu.async_copy(qsrc_hbm.at[pl.ds(qoff, CAP)], qs_v, sem0)
        pltpu.async_copy(qdst_hbm.at[pl.ds(qoff, CAP)], qd_v, sem1)
        pltpu.make_async_copy(qsrc_hbm.at[pl.ds(qoff, CAP)], qs_v, sem0).wait()
        pltpu.make_async_copy(qdst_hbm.at[pl.ds(qoff, CAP)], qd_v, sem1).wait()
        cidx = w * NW + t
        crow = cnt_v[pl.ds((cidx // 16) * 16, 16)]
        ncha = jnp.max(jnp.where(iota == cidx - (cidx // 16) * 16, crow, 0))

        @pl.when(ncha > 0)
        def _():
            pltpu.async_copy(h_hbm.at[qs_v.at[pl.ds(0, 16)]],
                             rows_v.at[pl.ds(0, 16)], sem0)

        def g_body(g, carry):
            nxt = g + 1

            @pl.when(nxt < ncha)
            def _():
                @pl.when(lax.rem(nxt, 2) == 0)
                def _():
                    pltpu.async_copy(h_hbm.at[qs_v.at[pl.ds(nxt * 16, 16)]],
                                     rows_v.at[pl.ds(0, 16)], sem0)

                @pl.when(lax.rem(nxt, 2) == 1)
                def _():
                    pltpu.async_copy(h_hbm.at[qs_v.at[pl.ds(nxt * 16, 16)]],
                                     rows_v.at[pl.ds(16, 16)], sem1)

            par = lax.rem(g, 2)

            @pl.when(par == 0)
            def _():
                pltpu.make_async_copy(h_hbm.at[qs_v.at[pl.ds(0, 16)]],
                                      rows_v.at[pl.ds(0, 16)], sem0).wait()

            @pl.when(par == 1)
            def _():
                pltpu.make_async_copy(h_hbm.at[qs_v.at[pl.ds(0, 16)]],
                                      rows_v.at[pl.ds(16, 16)], sem1).wait()

            dvec = qd_v[pl.ds(g * 16, 16)]
            rbase = par * 16
            for e in range(16):
                d_e = dvec[e]
                r = rbase + e
                for j in range(8):
                    sl = pl.ds(j * 16, 16)
                    acc_v[d_e, sl] = jnp.maximum(acc_v[d_e, sl], rows_v[r, sl])
            return carry

        lax.fori_loop(0, ncha, g_body, 0)
        return carry

    lax.fori_loop(0, NW, w_body, 0)
    pltpu.async_copy(acc_v.at[pl.ds(0, RPT)], out_hbm.at[pl.ds(t * RPT, RPT)],
                     sem0).wait()


@functools.partial(
    pl.kernel,
    out_type=jax.ShapeDtypeStruct((NPAD, 128), jnp.float32),
    mesh=_mesh(),
    scratch_types=(pltpu.VMEM((RPT + 8, 128), jnp.float32),
                   pltpu.VMEM((CAP,), jnp.int32),
                   pltpu.VMEM((CAP,), jnp.int32),
                   pltpu.VMEM((32, 128), jnp.float32),
                   pltpu.VMEM((NW * NW,), jnp.int32),
                   pltpu.SemaphoreType.DMA,
                   pltpu.SemaphoreType.DMA),
    compiler_params=_sc_params,
)
def _segmax(h_hbm, qsrc_hbm, qdst_hbm, counts_hbm, out_hbm, *rest):
    _segmax_body(h_hbm, qsrc_hbm, qdst_hbm, counts_hbm, out_hbm, *rest)


def _mm1_kernel(x_ref, w_ref, o_ref):
    o_ref[...] = jnp.dot(x_ref[...], w_ref[...],
                         preferred_element_type=jnp.float32)


def _matmul1(x, w):
    n, k = x.shape
    _, m = w.shape
    grid = pl.cdiv(n, ROW_BLK)
    return pl.pallas_call(
        _mm1_kernel,
        grid=(grid,),
        in_specs=[
            pl.BlockSpec((ROW_BLK, k), lambda i: (i, 0)),
            pl.BlockSpec((k, m), lambda i: (0, 0)),
        ],
        out_specs=pl.BlockSpec((ROW_BLK, m), lambda i: (i, 0)),
        out_shape=jax.ShapeDtypeStruct((n, m), jnp.float32),
    )(x, w)


def _relu_mm_kernel(x_ref, b_ref, w_ref, o_ref):
    h = jnp.maximum(x_ref[...] + b_ref[...], 0.0)
    o_ref[...] = jnp.dot(h, w_ref[...], preferred_element_type=jnp.float32)


def _relu_matmul(x, b, w):
    n, k = x.shape
    _, m = w.shape
    grid = pl.cdiv(n, ROW_BLK)
    return pl.pallas_call(
        _relu_mm_kernel,
        grid=(grid,),
        in_specs=[
            pl.BlockSpec((ROW_BLK, k), lambda i: (i, 0)),
            pl.BlockSpec((1, k), lambda i: (0, 0)),
            pl.BlockSpec((k, m), lambda i: (0, 0)),
        ],
        out_specs=pl.BlockSpec((ROW_BLK, m), lambda i: (i, 0)),
        out_shape=jax.ShapeDtypeStruct((n, m), jnp.float32),
    )(x, b.reshape(1, k), w)


def _tail_kernel(x_ref, h_ref, b_ref, o_ref):
    o_ref[...] = jnp.tanh(x_ref[:, :128] + h_ref[...] + b_ref[...])


def _tail(x, h, b):
    n = N_NODES
    m = 128
    blk = 1000
    return pl.pallas_call(
        _tail_kernel,
        grid=(n // blk,),
        in_specs=[
            pl.BlockSpec((blk, x.shape[1]), lambda i: (i, 0)),
            pl.BlockSpec((blk, m), lambda i: (i, 0)),
            pl.BlockSpec((1, m), lambda i: (0, 0)),
        ],
        out_specs=pl.BlockSpec((blk, m), lambda i: (i, 0)),
        out_shape=jax.ShapeDtypeStruct((n, m), jnp.float32),
    )(x, h, b.reshape(1, m))


def kernel(x, edge_index, W1, b1, W2, b2):
    xp = jnp.pad(x, ((0, NPAD - N_NODES), (0, 0)))
    qsrc, qdst, counts = _bucket_edges(edge_index.reshape(-1))
    ssrc, gd, tot = _merge_sort(qsrc, qdst, counts)
    h1 = _matmul1(xp, W1)
    m1 = _segmax2(h1, ssrc, gd, tot)
    h2 = _relu_matmul(m1, b1, W2)
    m2 = _segmax2(h2, ssrc, gd, tot)
    return _tail(x, m2, b2)


# speedup vs baseline: 8.7301x; 1.0388x over previous
import functools

import jax
import jax.numpy as jnp
from jax import lax
from jax.experimental import pallas as pl
from jax.experimental.pallas import tpu as pltpu
from jax.experimental.pallas import tpu_sc as plsc

N_NODES = 10000
N_EDGES = 320000
NW = 32
RPT = 320
NPAD = NW * RPT
EPW = N_EDGES // NW
CAP = 512
QTOT = NW * NW * CAP
DUMMY = RPT

ROW_BLK = 1024

_mesh = functools.partial(
    plsc.VectorSubcoreMesh, core_axis_name="c", subcore_axis_name="s",
    num_cores=2, num_subcores=16)

_sc_params = pltpu.CompilerParams(needs_layout_passes=False)


def _iota16():
    return lax.iota(jnp.int32, 16)


def _vgather(v, idx):
    return v.at[idx].get(mode="promise_in_bounds")


def _lane(v, e):
    return jnp.max(jnp.where(_iota16() == e, v, 0))


def _wid():
    return lax.axis_index("s") * 2 + lax.axis_index("c")


def _bucket_body(edge_hbm, qsrc_hbm, qdst_hbm, counts_hbm,
                 es_v, ed_v, qs_v, qd_v, cnt_v, ccnt_v, sem):
    w = _wid()
    pltpu.async_copy(edge_hbm.at[pl.ds(w * EPW, EPW)], es_v, sem).wait()
    pltpu.async_copy(edge_hbm.at[pl.ds(N_EDGES + w * EPW, EPW)], ed_v,
                     sem).wait()

    iota = _iota16()
    zeros = jnp.zeros((16,), jnp.int32)
    dummyv = jnp.full((16,), DUMMY, jnp.int32)

    cnt_v[pl.ds(0, 16)] = zeros
    cnt_v[pl.ds(16, 16)] = zeros

    def prefill(r, carry):
        base = r * 256
        for k in range(16):
            qd_v[pl.ds(base + k * 16, 16)] = dummyv
            qs_v[pl.ds(base + k * 16, 16)] = zeros
        return carry
    lax.fori_loop(0, NW * CAP // 256, prefill, 0)

    def body(i, carry):
        s = es_v[pl.ds(i * 16, 16)]
        d = ed_v[pl.ds(i * 16, 16)]
        b = ((d >> 6) * 205) >> 10
        bs, perm = plsc.sort_key_val(b, iota)
        ss = _vgather(s, perm)
        ds = _vgather(d, perm)
        prev = _vgather(bs, jnp.maximum(iota - 1, 0))
        rs = (iota == 0) | (bs != prev)
        sidx = plsc.cummax(jnp.where(rs, iota, 0))
        rank = iota - sidx
        base = plsc.load_gather(cnt_v, [bs])
        pos = base + rank
        nxt = _vgather(rs.astype(jnp.int32), jnp.minimum(iota + 1, 15))
        ls = (iota == 15) | ((iota < 15) & (nxt == 1))
        plsc.store_scatter(qs_v, [bs * CAP + pos], ss)
        plsc.store_scatter(qd_v, [bs * CAP + pos], ds - bs * RPT)
        plsc.store_scatter(cnt_v, [bs], pos + 1, mask=ls)
        return carry
    lax.fori_loop(0, EPW // 16, body, 0)

    ccnt_v[pl.ds(0, 16)] = (cnt_v[pl.ds(0, 16)] + 15) >> 4
    ccnt_v[pl.ds(16, 16)] = (cnt_v[pl.ds(16, 16)] + 15) >> 4

    pltpu.async_copy(qs_v, qsrc_hbm.at[pl.ds(w * NW * CAP, NW * CAP)],
                     sem).wait()
    pltpu.async_copy(qd_v, qdst_hbm.at[pl.ds(w * NW * CAP, NW * CAP)],
                     sem).wait()
    pltpu.async_copy(ccnt_v, counts_hbm.at[pl.ds(w * NW, NW)], sem).wait()


@functools.partial(
    pl.kernel,
    out_type=(jax.ShapeDtypeStruct((QTOT,), jnp.int32),
              jax.ShapeDtypeStruct((QTOT,), jnp.int32),
              jax.ShapeDtypeStruct((NW * NW,), jnp.int32)),
    mesh=_mesh(),
    scratch_types=(pltpu.VMEM((EPW,), jnp.int32),
                   pltpu.VMEM((EPW,), jnp.int32),
                   pltpu.VMEM((NW * CAP,), jnp.int32),
                   pltpu.VMEM((NW * CAP,), jnp.int32),
                   pltpu.VMEM((NW,), jnp.int32),
                   pltpu.VMEM((NW,), jnp.int32),
                   pltpu.SemaphoreType.DMA),
    compiler_params=_sc_params,
)
def _bucket_edges(edge_hbm, qsrc_hbm, qdst_hbm, counts_hbm, *rest):
    _bucket_body(edge_hbm, qsrc_hbm, qdst_hbm, counts_hbm, *rest)


SCAP = 20480
GCAP = 2560
WG = 24
WROWS = WG * 8


def _merge_sort_body(qsrc_hbm, qdst_hbm, counts_hbm,
                     ssrc_hbm, gd_hbm, tot_hbm,
                     qs_v, qd_v, bins_v, off_v, ng_v, cnt2_v,
                     ssrc_v, gd_v, tot_v, cntk_v, sem):
    t = _wid()
    iota = _iota16()
    zeros = jnp.zeros((16,), jnp.int32)

    def stage_issue(w, carry):
        qoff = (w * NW + t) * CAP
        pltpu.async_copy(qsrc_hbm.at[pl.ds(qoff, CAP)],
                         qs_v.at[pl.ds(w * CAP, CAP)], sem)
        pltpu.async_copy(qdst_hbm.at[pl.ds(qoff, CAP)],
                         qd_v.at[pl.ds(w * CAP, CAP)], sem)
        return carry
    lax.fori_loop(0, NW, stage_issue, 0)
    pltpu.async_copy(counts_hbm, cntk_v, sem)

    def stage_drain(w, carry):
        qoff = (w * NW + t) * CAP
        pltpu.make_async_copy(qsrc_hbm.at[pl.ds(qoff, CAP)],
                              qs_v.at[pl.ds(w * CAP, CAP)], sem).wait()
        pltpu.make_async_copy(qdst_hbm.at[pl.ds(qoff, CAP)],
                              qd_v.at[pl.ds(w * CAP, CAP)], sem).wait()
        return carry
    lax.fori_loop(0, NW, stage_drain, 0)
    pltpu.make_async_copy(counts_hbm, cntk_v, sem).wait()

    def _nch(w):
        row = cntk_v[pl.ds(w * NW + (t // 16) * 16, 16)]
        return jnp.max(jnp.where(iota == t - (t // 16) * 16, row, 0))

    for k in range(328 // 8 // 2):
        bins_v[pl.ds(k * 16, 16)] = zeros
        cnt2_v[pl.ds(k * 16, 16)] = zeros
    bins_v[pl.ds(328 - 16, 16)] = zeros
    cnt2_v[pl.ds(328 - 16, 16)] = zeros

    neg1 = jnp.full((16,), -1, jnp.int32)
    dum = jnp.full((16,), DUMMY, jnp.int32)

    def pre1(i, carry):
        ssrc_v[pl.ds(i * 16, 16)] = neg1
        return carry
    lax.fori_loop(0, SCAP // 16, pre1, 0)

    def pre2(i, carry):
        gd_v[pl.ds(i * 16, 16)] = dum
        return carry
    lax.fori_loop(0, GCAP // 16, pre2, 0)

    def hist_w(w, carry):
        qbase = w * CAP

        def hist(j, c):
            d = qd_v[pl.ds(qbase + j * 16, 16)]
            ds, _ = plsc.sort_key_val(d, iota)
            prev = _vgather(ds, jnp.maximum(iota - 1, 0))
            rs = (iota == 0) | (ds != prev)
            sidx = plsc.cummax(jnp.where(rs, iota, 0))
            rank = iota - sidx
            nxt = _vgather(rs.astype(jnp.int32), jnp.minimum(iota + 1, 15))
            ls = (iota == 15) | ((iota < 15) & (nxt == 1))
            base = plsc.load_gather(bins_v, [ds])
            plsc.store_scatter(bins_v, [ds], base + rank + 1, mask=ls)
            return c
        lax.fori_loop(0, _nch(w), hist, 0)
        return carry
    lax.fori_loop(0, NW, hist_w, 0)

    def prefix(v, carry):
        b = bins_v[pl.ds(v * 16, 16)]
        pad8 = (b + 7) & ~7
        cs = plsc.cumsum(pad8)
        off_v[pl.ds(v * 16, 16)] = carry + cs - pad8
        ng_v[pl.ds(v * 16, 16)] = pad8 >> 3
        return carry + cs[15]
    carry = lax.fori_loop(0, 320 // 16, prefix, 0)

    def gdfill(v, carry):
        offv = off_v[pl.ds(v * 16, 16)]
        ngv = ng_v[pl.ds(v * 16, 16)]
        for lane in range(16):
            goff = offv[lane] >> 3
            n = ngv[lane]
            dval = jnp.full((16,), v * 16 + lane, jnp.int32)

            def put(k, c):
                plsc.store_scatter(gd_v, [jnp.where(iota == 0, goff + k, 0)],
                                   dval, mask=(iota == 0))
                return c
            lax.fori_loop(0, n, put, 0)
        return carry
    lax.fori_loop(0, 20, gdfill, 0)

    def place_w(w, carry):
        qbase = w * CAP

        def place(j, c):
            s = qs_v[pl.ds(qbase + j * 16, 16)]
            d = qd_v[pl.ds(qbase + j * 16, 16)]
            ds, perm = plsc.sort_key_val(d, iota)
            ss = _vgather(s, perm)
            prev = _vgather(ds, jnp.maximum(iota - 1, 0))
            rs = (iota == 0) | (ds != prev)
            sidx = plsc.cummax(jnp.where(rs, iota, 0))
            rank = iota - sidx
            nxt = _vgather(rs.astype(jnp.int32), jnp.minimum(iota + 1, 15))
            ls = (iota == 15) | ((iota < 15) & (nxt == 1))
            valid = ds != DUMMY
            base = plsc.load_gather(cnt2_v, [ds])
            tgt = plsc.load_gather(off_v, [jnp.minimum(ds, 319)])
            plsc.store_scatter(ssrc_v,
                               [jnp.minimum(tgt + base + rank, SCAP - 1)],
                               ss, mask=valid)
            plsc.store_scatter(cnt2_v, [ds], base + rank + 1, mask=ls)
            return c
        lax.fori_loop(0, _nch(w), place, 0)
        return carry
    lax.fori_loop(0, NW, place_w, 0)

    g_tot = carry >> 3
    nwin = ((g_tot + WG - 1) * 2731) >> 16
    tot_v[pl.ds(0, 16)] = jnp.where(iota == 0, nwin, 0)

    pltpu.async_copy(ssrc_v, ssrc_hbm.at[pl.ds(t * SCAP, SCAP)], sem).wait()
    pltpu.async_copy(gd_v, gd_hbm.at[pl.ds(t * GCAP, GCAP)], sem).wait()
    pltpu.async_copy(tot_v, tot_hbm.at[pl.ds(t * 16, 16)], sem).wait()


@functools.partial(
    pl.kernel,
    out_type=(jax.ShapeDtypeStruct((NW * SCAP,), jnp.int32),
              jax.ShapeDtypeStruct((NW * GCAP,), jnp.int32),
              jax.ShapeDtypeStruct((NW * 16,), jnp.int32)),
    mesh=_mesh(),
    scratch_types=(pltpu.VMEM((NW * CAP,), jnp.int32),
                   pltpu.VMEM((NW * CAP,), jnp.int32),
                   pltpu.VMEM((328,), jnp.int32),
                   pltpu.VMEM((328,), jnp.int32),
                   pltpu.VMEM((320,), jnp.int32),
                   pltpu.VMEM((328,), jnp.int32),
                   pltpu.VMEM((SCAP,), jnp.int32),
                   pltpu.VMEM((GCAP,), jnp.int32),
                   pltpu.VMEM((16,), jnp.int32),
                   pltpu.VMEM((NW * NW,), jnp.int32),
                   pltpu.SemaphoreType.DMA),
    compiler_params=_sc_params,
)
def _merge_sort(qsrc_hbm, qdst_hbm, counts_hbm, ssrc_hbm, gd_hbm, tot_hbm,
                *rest):
    _merge_sort_body(qsrc_hbm, qdst_hbm, counts_hbm, ssrc_hbm, gd_hbm,
                     tot_hbm, *rest)


def _segmax2_body(h_hbm, ssrc_hbm, gd_hbm, tot_hbm, out_hbm,
                  acc_v, gd_v, rows_v, idx_v, tot_v,
                  semg0, semg1, semg2, semi0, semi1, semi2):
    t = _wid()
    iota = _iota16()
    semg = (semg0, semg1, semg2)
    semi = (semi0, semi1, semi2)

    pltpu.async_copy(h_hbm.at[pl.ds(t * RPT, RPT)], acc_v.at[pl.ds(0, RPT)],
                     semg0)
    pltpu.async_copy(gd_hbm.at[pl.ds(t * GCAP, GCAP)], gd_v, semg1)
    pltpu.async_copy(tot_hbm.at[pl.ds(t * 16, 16)], tot_v, semg2)
    pltpu.make_async_copy(h_hbm.at[pl.ds(t * RPT, RPT)],
                          acc_v.at[pl.ds(0, RPT)], semg0).wait()
    pltpu.make_async_copy(gd_hbm.at[pl.ds(t * GCAP, GCAP)], gd_v,
                          semg1).wait()
    pltpu.make_async_copy(tot_hbm.at[pl.ds(t * 16, 16)], tot_v, semg2).wait()
    nwin = tot_v[pl.ds(0, 16)][0]

    def idx_issue(w, s):
        pltpu.async_copy(ssrc_hbm.at[pl.ds(t * SCAP + w * WROWS, WROWS)],
                         idx_v.at[pl.ds(s * WROWS, WROWS)], semi[s])

    def idx_wait(s):
        pltpu.make_async_copy(ssrc_hbm.at[pl.ds(t * SCAP, WROWS)],
                              idx_v.at[pl.ds(s * WROWS, WROWS)],
                              semi[s]).wait()

    def fix_slot(w, s):
        for v in range(WROWS // 16):
            sl = pl.ds(s * WROWS + v * 16, 16)
            sv = idx_v[sl]
            gdl = plsc.load_gather(gd_v, [w * WG + ((v * 16 + iota) >> 3)])
            selfn = jnp.minimum(t * RPT + gdl, NPAD - 1)
            idx_v[sl] = jnp.where(sv < 0, selfn, sv)

    def gather_issue(s):
        base = s * WROWS
        pltpu.async_copy(h_hbm.at[idx_v.at[pl.ds(base, 96)]],
                         rows_v.at[pl.ds(base, 96)], semg[s])
        pltpu.async_copy(h_hbm.at[idx_v.at[pl.ds(base + 96, 96)]],
                         rows_v.at[pl.ds(base + 96, 96)], semg[s])

    def gather_wait(s):
        base = s * WROWS
        pltpu.make_async_copy(h_hbm.at[idx_v.at[pl.ds(base, 96)]],
                              rows_v.at[pl.ds(base, 96)], semg[s]).wait()
        pltpu.make_async_copy(h_hbm.at[idx_v.at[pl.ds(base, 96)]],
                              rows_v.at[pl.ds(base + 96, 96)], semg[s]).wait()

    for s in range(2):
        @pl.when(nwin > s)
        def _(s=s):
            idx_issue(s, s)
            idx_wait(s)
            fix_slot(s, s)
            gather_issue(s)

    @pl.when(nwin > 2)
    def _():
        idx_issue(2, 2)

    def w_body(w, carry):
        p0 = lax.rem(w, 3)
        p2 = lax.rem(w + 2, 3)

        @pl.when(w + 2 < nwin)
        def _():
            for s in range(3):
                @pl.when(p2 == s)
                def _(s=s):
                    idx_wait(s)
                    fix_slot(w + 2, s)
                    gather_issue(s)

        for s in range(3):
            @pl.when(p0 == s)
            def _(s=s):
                gather_wait(s)

        @pl.when(w + 3 < nwin)
        def _():
            for s in range(3):
                @pl.when(p0 == s)
                def _(s=s):
                    idx_issue(w + 3, s)

        gd0 = gd_v[pl.ds(w * WG, 16)]
        gd1 = gd_v[pl.ds(w * WG + 8, 16)]
        rbase = p0 * WROWS
        for grp in range(WG):
            if grp < 16:
                d_g = gd0[grp]
            else:
                d_g = gd1[grp - 8]
            rb = rbase + grp * 8
            ms = []
            for j in range(8):
                sl = pl.ds(j * 16, 16)
                m0 = jnp.maximum(rows_v[rb, sl], rows_v[rb + 1, sl])
                m1 = jnp.maximum(rows_v[rb + 2, sl], rows_v[rb + 3, sl])
                m2 = jnp.maximum(rows_v[rb + 4, sl], rows_v[rb + 5, sl])
                m3 = jnp.maximum(rows_v[rb + 6, sl], rows_v[rb + 7, sl])
                ms.append(jnp.maximum(jnp.maximum(m0, m1),
                                      jnp.maximum(m2, m3)))
            for j in range(8):
                sl = pl.ds(j * 16, 16)
                acc_v[d_g, sl] = jnp.maximum(acc_v[d_g, sl], ms[j])
        return carry

    lax.fori_loop(0, nwin, w_body, 0)
    pltpu.async_copy(acc_v.at[pl.ds(0, RPT)], out_hbm.at[pl.ds(t * RPT, RPT)],
                     semg0).wait()


@functools.partial(
    pl.kernel,
    out_type=jax.ShapeDtypeStruct((NPAD, 128), jnp.float32),
    mesh=_mesh(),
    scratch_types=(pltpu.VMEM((RPT + 8, 128), jnp.float32),
                   pltpu.VMEM((GCAP,), jnp.int32),
                   pltpu.VMEM((3 * WROWS, 128), jnp.float32),
                   pltpu.VMEM((3 * WROWS,), jnp.int32),
                   pltpu.VMEM((16,), jnp.int32),
                   pltpu.SemaphoreType.DMA,
                   pltpu.SemaphoreType.DMA,
                   pltpu.SemaphoreType.DMA,
                   pltpu.SemaphoreType.DMA,
                   pltpu.SemaphoreType.DMA,
                   pltpu.SemaphoreType.DMA),
    compiler_params=_sc_params,
)
def _segmax2(h_hbm, ssrc_hbm, gd_hbm, tot_hbm, out_hbm, *rest):
    _segmax2_body(h_hbm, ssrc_hbm, gd_hbm, tot_hbm, out_hbm, *rest)


def _segmax_body(h_hbm, qsrc_hbm, qdst_hbm, counts_hbm, out_hbm,
                 acc_v, qs_v, qd_v, rows_v, cnt_v, sem0, sem1):
    t = _wid()
    pltpu.async_copy(h_hbm.at[pl.ds(t * RPT, RPT)], acc_v.at[pl.ds(0, RPT)],
                     sem0).wait()
    pltpu.async_copy(counts_hbm, cnt_v, sem0).wait()

    iota = _iota16()

    def w_body(w, carry):
        qoff = (w * NW + t) * CAP
        pltpu.async_copy(qsrc_hbm.at[pl.ds(qoff, CAP)], qs_v, sem0)
        pltpu.async_copy(qdst_hbm.at[pl.ds(qoff, CAP)], qd_v, sem1)
        pltpu.make_async_copy(qsrc_hbm.at[pl.ds(qoff, CAP)], qs_v, sem0).wait()
        pltpu.make_async_copy(qdst_hbm.at[pl.ds(qoff, CAP)], qd_v, sem1).wait()
        cidx = w * NW + t
        crow = cnt_v[pl.ds((cidx // 16) * 16, 16)]
        ncha = jnp.max(jnp.where(iota == cidx - (cidx // 16) * 16, crow, 0))

        @pl.when(ncha > 0)
        def _():
            pltpu.async_copy(h_hbm.at[qs_v.at[pl.ds(0, 16)]],
                             rows_v.at[pl.ds(0, 16)], sem0)

        def g_body(g, carry):
            nxt = g + 1

            @pl.when(nxt < ncha)
            def _():
                @pl.when(lax.rem(nxt, 2) == 0)
                def _():
                    pltpu.async_copy(h_hbm.at[qs_v.at[pl.ds(nxt * 16, 16)]],
                                     rows_v.at[pl.ds(0, 16)], sem0)

                @pl.when(lax.rem(nxt, 2) == 1)
                def _():
                    pltpu.async_copy(h_hbm.at[qs_v.at[pl.ds(nxt * 16, 16)]],
                                     rows_v.at[pl.ds(16, 16)], sem1)

            par = lax.rem(g, 2)

            @pl.when(par == 0)
            def _():
                pltpu.make_async_copy(h_hbm.at[qs_v.at[pl.ds(0, 16)]],
                                      rows_v.at[pl.ds(0, 16)], sem0).wait()

            @pl.when(par == 1)
            def _():
                pltpu.make_async_copy(h_hbm.at[qs_v.at[pl.ds(0, 16)]],
                                      rows_v.at[pl.ds(16, 16)], sem1).wait()

            dvec = qd_v[pl.ds(g * 16, 16)]
            rbase = par * 16
            for e in range(16):
                d_e = dvec[e]
                r = rbase + e
                for j in range(8):
                    sl = pl.ds(j * 16, 16)
                    acc_v[d_e, sl] = jnp.maximum(acc_v[d_e, sl], rows_v[r, sl])
            return carry

        lax.fori_loop(0, ncha, g_body, 0)
        return carry

    lax.fori_loop(0, NW, w_body, 0)
    pltpu.async_copy(acc_v.at[pl.ds(0, RPT)], out_hbm.at[pl.ds(t * RPT, RPT)],
                     sem0).wait()


@functools.partial(
    pl.kernel,
    out_type=jax.ShapeDtypeStruct((NPAD, 128), jnp.float32),
    mesh=_mesh(),
    scratch_types=(pltpu.VMEM((RPT + 8, 128), jnp.float32),
                   pltpu.VMEM((CAP,), jnp.int32),
                   pltpu.VMEM((CAP,), jnp.int32),
                   pltpu.VMEM((32, 128), jnp.float32),
                   pltpu.VMEM((NW * NW,), jnp.int32),
                   pltpu.SemaphoreType.DMA,
                   pltpu.SemaphoreType.DMA),
    compiler_params=_sc_params,
)
def _segmax(h_hbm, qsrc_hbm, qdst_hbm, counts_hbm, out_hbm, *rest):
    _segmax_body(h_hbm, qsrc_hbm, qdst_hbm, counts_hbm, out_hbm, *rest)


def _mm1_kernel(x_ref, w_ref, o_ref):
    o_ref[...] = jnp.dot(x_ref[...], w_ref[...],
                         preferred_element_type=jnp.float32)


def _matmul1(x, w):
    n, k = x.shape
    _, m = w.shape
    grid = pl.cdiv(n, ROW_BLK)
    return pl.pallas_call(
        _mm1_kernel,
        grid=(grid,),
        in_specs=[
            pl.BlockSpec((ROW_BLK, k), lambda i: (i, 0)),
            pl.BlockSpec((k, m), lambda i: (0, 0)),
        ],
        out_specs=pl.BlockSpec((ROW_BLK, m), lambda i: (i, 0)),
        out_shape=jax.ShapeDtypeStruct((n, m), jnp.float32),
    )(x, w)


def _relu_mm_kernel(x_ref, b_ref, w_ref, o_ref):
    h = jnp.maximum(x_ref[...] + b_ref[...], 0.0)
    o_ref[...] = jnp.dot(h, w_ref[...], preferred_element_type=jnp.float32)


def _relu_matmul(x, b, w):
    n, k = x.shape
    _, m = w.shape
    grid = pl.cdiv(n, ROW_BLK)
    return pl.pallas_call(
        _relu_mm_kernel,
        grid=(grid,),
        in_specs=[
            pl.BlockSpec((ROW_BLK, k), lambda i: (i, 0)),
            pl.BlockSpec((1, k), lambda i: (0, 0)),
            pl.BlockSpec((k, m), lambda i: (0, 0)),
        ],
        out_specs=pl.BlockSpec((ROW_BLK, m), lambda i: (i, 0)),
        out_shape=jax.ShapeDtypeStruct((n, m), jnp.float32),
    )(x, b.reshape(1, k), w)


def _tail_kernel(x_ref, h_ref, b_ref, o_ref):
    o_ref[...] = jnp.tanh(x_ref[:, :128] + h_ref[...] + b_ref[...])


def _tail(x, h, b):
    n = N_NODES
    m = 128
    blk = 1000
    return pl.pallas_call(
        _tail_kernel,
        grid=(n // blk,),
        in_specs=[
            pl.BlockSpec((blk, x.shape[1]), lambda i: (i, 0)),
            pl.BlockSpec((blk, m), lambda i: (i, 0)),
            pl.BlockSpec((1, m), lambda i: (0, 0)),
        ],
        out_specs=pl.BlockSpec((blk, m), lambda i: (i, 0)),
        out_shape=jax.ShapeDtypeStruct((n, m), jnp.float32),
    )(x, h, b.reshape(1, m))


def kernel(x, edge_index, W1, b1, W2, b2):
    xp = jnp.pad(x, ((0, NPAD - N_NODES), (0, 0)))
    qsrc, qdst, counts = _bucket_edges(edge_index.reshape(-1))
    ssrc, gd, tot = _merge_sort(qsrc, qdst, counts)
    h1 = _matmul1(xp, W1)
    m1 = _segmax2(h1, ssrc, gd, tot)
    h2 = _relu_matmul(m1, b1, W2)
    m2 = _segmax2(h2, ssrc, gd, tot)
    return _tail(x, m2, b2)


# speedup vs baseline: 9.8041x; 1.1230x over previous
import functools

import jax
import jax.numpy as jnp
from jax import lax
from jax.experimental import pallas as pl
from jax.experimental.pallas import tpu as pltpu
from jax.experimental.pallas import tpu_sc as plsc

N_NODES = 10000
N_EDGES = 320000
NW = 32
RPT = 320
NPAD = NW * RPT
EPW = N_EDGES // NW
CAP = 512
QTOT = NW * NW * CAP
DUMMY = RPT

ROW_BLK = 1024

_mesh = functools.partial(
    plsc.VectorSubcoreMesh, core_axis_name="c", subcore_axis_name="s",
    num_cores=2, num_subcores=16)

_sc_params = pltpu.CompilerParams(needs_layout_passes=False)


def _iota16():
    return lax.iota(jnp.int32, 16)


def _vgather(v, idx):
    return v.at[idx].get(mode="promise_in_bounds")


def _lane(v, e):
    return jnp.max(jnp.where(_iota16() == e, v, 0))


def _wid():
    return lax.axis_index("s") * 2 + lax.axis_index("c")


def _bucket_body(edge_hbm, qsrc_hbm, qdst_hbm, counts_hbm,
                 es_v, ed_v, qs_v, qd_v, cnt_v, ccnt_v, sem):
    w = _wid()
    pltpu.async_copy(edge_hbm.at[pl.ds(w * EPW, EPW)], es_v, sem).wait()
    pltpu.async_copy(edge_hbm.at[pl.ds(N_EDGES + w * EPW, EPW)], ed_v,
                     sem).wait()

    iota = _iota16()
    zeros = jnp.zeros((16,), jnp.int32)
    dummyv = jnp.full((16,), DUMMY, jnp.int32)

    cnt_v[pl.ds(0, 16)] = zeros
    cnt_v[pl.ds(16, 16)] = zeros

    def prefill(r, carry):
        base = r * 256
        for k in range(16):
            qd_v[pl.ds(base + k * 16, 16)] = dummyv
            qs_v[pl.ds(base + k * 16, 16)] = zeros
        return carry
    lax.fori_loop(0, NW * CAP // 256, prefill, 0)

    def body(i, carry):
        s = es_v[pl.ds(i * 16, 16)]
        d = ed_v[pl.ds(i * 16, 16)]
        b = ((d >> 6) * 205) >> 10
        bs, perm = plsc.sort_key_val(b, iota)
        ss = _vgather(s, perm)
        ds = _vgather(d, perm)
        prev = _vgather(bs, jnp.maximum(iota - 1, 0))
        rs = (iota == 0) | (bs != prev)
        sidx = plsc.cummax(jnp.where(rs, iota, 0))
        rank = iota - sidx
        base = plsc.load_gather(cnt_v, [bs])
        pos = base + rank
        nxt = _vgather(rs.astype(jnp.int32), jnp.minimum(iota + 1, 15))
        ls = (iota == 15) | ((iota < 15) & (nxt == 1))
        plsc.store_scatter(qs_v, [bs * CAP + pos], ss)
        plsc.store_scatter(qd_v, [bs * CAP + pos], ds - bs * RPT)
        plsc.store_scatter(cnt_v, [bs], pos + 1, mask=ls)
        return carry
    lax.fori_loop(0, EPW // 16, body, 0)

    ccnt_v[pl.ds(0, 16)] = (cnt_v[pl.ds(0, 16)] + 15) >> 4
    ccnt_v[pl.ds(16, 16)] = (cnt_v[pl.ds(16, 16)] + 15) >> 4

    pltpu.async_copy(qs_v, qsrc_hbm.at[pl.ds(w * NW * CAP, NW * CAP)],
                     sem).wait()
    pltpu.async_copy(qd_v, qdst_hbm.at[pl.ds(w * NW * CAP, NW * CAP)],
                     sem).wait()
    pltpu.async_copy(ccnt_v, counts_hbm.at[pl.ds(w * NW, NW)], sem).wait()


@functools.partial(
    pl.kernel,
    out_type=(jax.ShapeDtypeStruct((QTOT,), jnp.int32),
              jax.ShapeDtypeStruct((QTOT,), jnp.int32),
              jax.ShapeDtypeStruct((NW * NW,), jnp.int32)),
    mesh=_mesh(),
    scratch_types=(pltpu.VMEM((EPW,), jnp.int32),
                   pltpu.VMEM((EPW,), jnp.int32),
                   pltpu.VMEM((NW * CAP,), jnp.int32),
                   pltpu.VMEM((NW * CAP,), jnp.int32),
                   pltpu.VMEM((NW,), jnp.int32),
                   pltpu.VMEM((NW,), jnp.int32),
                   pltpu.SemaphoreType.DMA),
    compiler_params=_sc_params,
)
def _bucket_edges(edge_hbm, qsrc_hbm, qdst_hbm, counts_hbm, *rest):
    _bucket_body(edge_hbm, qsrc_hbm, qdst_hbm, counts_hbm, *rest)


SCAP = 20480
GCAP = 2560
WG = 24
WROWS = WG * 8


def _merge_sort_body(qsrc_hbm, qdst_hbm, counts_hbm,
                     ssrc_hbm, gd_hbm, tot_hbm,
                     qs_v, qd_v, bins_v, off_v, ng_v, cnt2_v,
                     ssrc_v, gd_v, tot_v, cntk_v, sem):
    t = _wid()
    iota = _iota16()
    zeros = jnp.zeros((16,), jnp.int32)

    def stage_issue(w, carry):
        qoff = (w * NW + t) * CAP
        pltpu.async_copy(qsrc_hbm.at[pl.ds(qoff, CAP)],
                         qs_v.at[pl.ds(w * CAP, CAP)], sem)
        pltpu.async_copy(qdst_hbm.at[pl.ds(qoff, CAP)],
                         qd_v.at[pl.ds(w * CAP, CAP)], sem)
        return carry
    lax.fori_loop(0, NW, stage_issue, 0)
    pltpu.async_copy(counts_hbm, cntk_v, sem)

    def stage_drain(w, carry):
        qoff = (w * NW + t) * CAP
        pltpu.make_async_copy(qsrc_hbm.at[pl.ds(qoff, CAP)],
                              qs_v.at[pl.ds(w * CAP, CAP)], sem).wait()
        pltpu.make_async_copy(qdst_hbm.at[pl.ds(qoff, CAP)],
                              qd_v.at[pl.ds(w * CAP, CAP)], sem).wait()
        return carry
    lax.fori_loop(0, NW, stage_drain, 0)
    pltpu.make_async_copy(counts_hbm, cntk_v, sem).wait()

    def _nch(w):
        row = cntk_v[pl.ds(w * NW + (t // 16) * 16, 16)]
        return jnp.max(jnp.where(iota == t - (t // 16) * 16, row, 0))

    for k in range(328 // 8 // 2):
        bins_v[pl.ds(k * 16, 16)] = zeros
        cnt2_v[pl.ds(k * 16, 16)] = zeros
    bins_v[pl.ds(328 - 16, 16)] = zeros
    cnt2_v[pl.ds(328 - 16, 16)] = zeros

    neg1 = jnp.full((16,), -1, jnp.int32)
    dum = jnp.full((16,), DUMMY, jnp.int32)

    def pre1(i, carry):
        ssrc_v[pl.ds(i * 16, 16)] = neg1
        return carry
    lax.fori_loop(0, SCAP // 16, pre1, 0)

    def pre2(i, carry):
        gd_v[pl.ds(i * 16, 16)] = dum
        return carry
    lax.fori_loop(0, GCAP // 16, pre2, 0)

    def hist_w(w, carry):
        qbase = w * CAP

        def hist(j, c):
            d = qd_v[pl.ds(qbase + j * 16, 16)]
            ds, _ = plsc.sort_key_val(d, iota)
            prev = _vgather(ds, jnp.maximum(iota - 1, 0))
            rs = (iota == 0) | (ds != prev)
            sidx = plsc.cummax(jnp.where(rs, iota, 0))
            rank = iota - sidx
            nxt = _vgather(rs.astype(jnp.int32), jnp.minimum(iota + 1, 15))
            ls = (iota == 15) | ((iota < 15) & (nxt == 1))
            base = plsc.load_gather(bins_v, [ds])
            plsc.store_scatter(bins_v, [ds], base + rank + 1, mask=ls)
            return c
        lax.fori_loop(0, _nch(w), hist, 0)
        return carry
    lax.fori_loop(0, NW, hist_w, 0)

    def prefix(v, carry):
        b = bins_v[pl.ds(v * 16, 16)]
        pad8 = (b + 7) & ~7
        cs = plsc.cumsum(pad8)
        off_v[pl.ds(v * 16, 16)] = carry + cs - pad8
        ng_v[pl.ds(v * 16, 16)] = pad8 >> 3
        return carry + cs[15]
    carry = lax.fori_loop(0, 320 // 16, prefix, 0)

    def gdfill(v, carry):
        offv = off_v[pl.ds(v * 16, 16)]
        ngv = ng_v[pl.ds(v * 16, 16)]
        bv = bins_v[pl.ds(v * 16, 16)]
        m0 = iota == 0
        for lane in range(16):
            o = offv[lane]
            goff = o >> 3
            n = ngv[lane]
            c = bv[lane]
            dval = jnp.full((16,), v * 16 + lane, jnp.int32)

            def put(k, cc):
                plsc.store_scatter(gd_v, [jnp.where(m0, goff + k, 0)],
                                   dval, mask=m0)
                return cc
            lax.fori_loop(0, n, put, 0)

            selfv = dval + t * RPT

            def padput(k, cc):
                plsc.store_scatter(ssrc_v, [jnp.where(m0, o + k, 0)],
                                   selfv, mask=m0)
                return cc
            lax.fori_loop(c, n * 8, padput, 0)
        return carry
    lax.fori_loop(0, 20, gdfill, 0)

    def place_w(w, carry):
        qbase = w * CAP

        def place(j, c):
            s = qs_v[pl.ds(qbase + j * 16, 16)]
            d = qd_v[pl.ds(qbase + j * 16, 16)]
            ds, perm = plsc.sort_key_val(d, iota)
            ss = _vgather(s, perm)
            prev = _vgather(ds, jnp.maximum(iota - 1, 0))
            rs = (iota == 0) | (ds != prev)
            sidx = plsc.cummax(jnp.where(rs, iota, 0))
            rank = iota - sidx
            nxt = _vgather(rs.astype(jnp.int32), jnp.minimum(iota + 1, 15))
            ls = (iota == 15) | ((iota < 15) & (nxt == 1))
            valid = ds != DUMMY
            base = plsc.load_gather(cnt2_v, [ds])
            tgt = plsc.load_gather(off_v, [jnp.minimum(ds, 319)])
            plsc.store_scatter(ssrc_v,
                               [jnp.minimum(tgt + base + rank, SCAP - 1)],
                               ss, mask=valid)
            plsc.store_scatter(cnt2_v, [ds], base + rank + 1, mask=ls)
            return c
        lax.fori_loop(0, _nch(w), place, 0)
        return carry
    lax.fori_loop(0, NW, place_w, 0)

    g_tot = carry >> 3
    nwin = ((g_tot + WG - 1) * 2731) >> 16
    tot_v[pl.ds(0, 16)] = jnp.where(iota == 0, nwin, 0)

    dumself = jnp.zeros((16,), jnp.int32) + jnp.minimum(
        t * RPT + DUMMY, NPAD - 1)
    m0 = iota == 0

    def tailput(k, cc):
        plsc.store_scatter(ssrc_v, [jnp.where(m0, k, 0)], dumself, mask=m0)
        return cc
    lax.fori_loop(carry, nwin * (WG * 8), tailput, 0)

    pltpu.async_copy(ssrc_v, ssrc_hbm.at[pl.ds(t * SCAP, SCAP)], sem).wait()
    pltpu.async_copy(gd_v, gd_hbm.at[pl.ds(t * GCAP, GCAP)], sem).wait()
    pltpu.async_copy(tot_v, tot_hbm.at[pl.ds(t * 16, 16)], sem).wait()


@functools.partial(
    pl.kernel,
    out_type=(jax.ShapeDtypeStruct((NW * SCAP,), jnp.int32),
              jax.ShapeDtypeStruct((NW * GCAP,), jnp.int32),
              jax.ShapeDtypeStruct((NW * 16,), jnp.int32)),
    mesh=_mesh(),
    scratch_types=(pltpu.VMEM((NW * CAP,), jnp.int32),
                   pltpu.VMEM((NW * CAP,), jnp.int32),
                   pltpu.VMEM((328,), jnp.int32),
                   pltpu.VMEM((328,), jnp.int32),
                   pltpu.VMEM((320,), jnp.int32),
                   pltpu.VMEM((328,), jnp.int32),
                   pltpu.VMEM((SCAP,), jnp.int32),
                   pltpu.VMEM((GCAP,), jnp.int32),
                   pltpu.VMEM((16,), jnp.int32),
                   pltpu.VMEM((NW * NW,), jnp.int32),
                   pltpu.SemaphoreType.DMA),
    compiler_params=_sc_params,
)
def _merge_sort(qsrc_hbm, qdst_hbm, counts_hbm, ssrc_hbm, gd_hbm, tot_hbm,
                *rest):
    _merge_sort_body(qsrc_hbm, qdst_hbm, counts_hbm, ssrc_hbm, gd_hbm,
                     tot_hbm, *rest)


def _segmax2_body(h_hbm, ssrc_hbm, gd_hbm, tot_hbm, out_hbm,
                  acc_v, gd_v, rows_v, idx_v, tot_v,
                  semg0, semg1, semg2, semi0, semi1, semi2):
    t = _wid()
    iota = _iota16()
    semg = (semg0, semg1, semg2)
    semi = (semi0, semi1, semi2)

    pltpu.async_copy(h_hbm.at[pl.ds(t * RPT, RPT)], acc_v.at[pl.ds(0, RPT)],
                     semg0)
    pltpu.async_copy(gd_hbm.at[pl.ds(t * GCAP, GCAP)], gd_v, semg1)
    pltpu.async_copy(tot_hbm.at[pl.ds(t * 16, 16)], tot_v, semg2)
    pltpu.make_async_copy(h_hbm.at[pl.ds(t * RPT, RPT)],
                          acc_v.at[pl.ds(0, RPT)], semg0).wait()
    pltpu.make_async_copy(gd_hbm.at[pl.ds(t * GCAP, GCAP)], gd_v,
                          semg1).wait()
    pltpu.make_async_copy(tot_hbm.at[pl.ds(t * 16, 16)], tot_v, semg2).wait()
    nwin = tot_v[pl.ds(0, 16)][0]

    def idx_issue(w, s):
        pltpu.async_copy(ssrc_hbm.at[pl.ds(t * SCAP + w * WROWS, WROWS)],
                         idx_v.at[pl.ds(s * WROWS, WROWS)], semi[s])

    def idx_wait(s):
        pltpu.make_async_copy(ssrc_hbm.at[pl.ds(t * SCAP, WROWS)],
                              idx_v.at[pl.ds(s * WROWS, WROWS)],
                              semi[s]).wait()

    def gather_issue(s):
        base = s * WROWS
        pltpu.async_copy(h_hbm.at[idx_v.at[pl.ds(base, 96)]],
                         rows_v.at[pl.ds(base, 96)], semg[s])
        pltpu.async_copy(h_hbm.at[idx_v.at[pl.ds(base + 96, 96)]],
                         rows_v.at[pl.ds(base + 96, 96)], semg[s])

    def gather_wait(s):
        base = s * WROWS
        pltpu.make_async_copy(h_hbm.at[idx_v.at[pl.ds(base, 96)]],
                              rows_v.at[pl.ds(base, 96)], semg[s]).wait()
        pltpu.make_async_copy(h_hbm.at[idx_v.at[pl.ds(base, 96)]],
                              rows_v.at[pl.ds(base + 96, 96)], semg[s]).wait()

    for s in range(2):
        @pl.when(nwin > s)
        def _(s=s):
            idx_issue(s, s)
            idx_wait(s)
            gather_issue(s)

    @pl.when(nwin > 2)
    def _():
        idx_issue(2, 2)

    def w_body(w, carry):
        p0 = lax.rem(w, 3)
        p2 = lax.rem(w + 2, 3)

        @pl.when(w + 2 < nwin)
        def _():
            for s in range(3):
                @pl.when(p2 == s)
                def _(s=s):
                    idx_wait(s)
                    gather_issue(s)

        for s in range(3):
            @pl.when(p0 == s)
            def _(s=s):
                gather_wait(s)

        @pl.when(w + 3 < nwin)
        def _():
            for s in range(3):
                @pl.when(p0 == s)
                def _(s=s):
                    idx_issue(w + 3, s)

        gd0 = gd_v[pl.ds(w * WG, 16)]
        gd1 = gd_v[pl.ds(w * WG + 8, 16)]
        rbase = p0 * WROWS
        for grp in range(WG):
            if grp < 16:
                d_g = gd0[grp]
            else:
                d_g = gd1[grp - 8]
            rb = rbase + grp * 8
            ms = []
            for j in range(8):
                sl = pl.ds(j * 16, 16)
                m0 = jnp.maximum(rows_v[rb, sl], rows_v[rb + 1, sl])
                m1 = jnp.maximum(rows_v[rb + 2, sl], rows_v[rb + 3, sl])
                m2 = jnp.maximum(rows_v[rb + 4, sl], rows_v[rb + 5, sl])
                m3 = jnp.maximum(rows_v[rb + 6, sl], rows_v[rb + 7, sl])
                ms.append(jnp.maximum(jnp.maximum(m0, m1),
                                      jnp.maximum(m2, m3)))
            for j in range(8):
                sl = pl.ds(j * 16, 16)
                acc_v[d_g, sl] = jnp.maximum(acc_v[d_g, sl], ms[j])
        return carry

    lax.fori_loop(0, nwin, w_body, 0)
    pltpu.async_copy(acc_v.at[pl.ds(0, RPT)], out_hbm.at[pl.ds(t * RPT, RPT)],
                     semg0).wait()


@functools.partial(
    pl.kernel,
    out_type=jax.ShapeDtypeStruct((NPAD, 128), jnp.float32),
    mesh=_mesh(),
    scratch_types=(pltpu.VMEM((RPT + 8, 128), jnp.float32),
                   pltpu.VMEM((GCAP,), jnp.int32),
                   pltpu.VMEM((3 * WROWS, 128), jnp.float32),
                   pltpu.VMEM((3 * WROWS,), jnp.int32),
                   pltpu.VMEM((16,), jnp.int32),
                   pltpu.SemaphoreType.DMA,
                   pltpu.SemaphoreType.DMA,
                   pltpu.SemaphoreType.DMA,
                   pltpu.SemaphoreType.DMA,
                   pltpu.SemaphoreType.DMA,
                   pltpu.SemaphoreType.DMA),
    compiler_params=_sc_params,
)
def _segmax2(h_hbm, ssrc_hbm, gd_hbm, tot_hbm, out_hbm, *rest):
    _segmax2_body(h_hbm, ssrc_hbm, gd_hbm, tot_hbm, out_hbm, *rest)


def _mm1_kernel(x_ref, w_ref, o_ref):
    o_ref[...] = jnp.dot(x_ref[...], w_ref[...],
                         preferred_element_type=jnp.float32)


def _matmul1(x, w):
    n, k = x.shape
    _, m = w.shape
    grid = pl.cdiv(n, ROW_BLK)
    return pl.pallas_call(
        _mm1_kernel,
        grid=(grid,),
        in_specs=[
            pl.BlockSpec((ROW_BLK, k), lambda i: (i, 0)),
            pl.BlockSpec((k, m), lambda i: (0, 0)),
        ],
        out_specs=pl.BlockSpec((ROW_BLK, m), lambda i: (i, 0)),
        out_shape=jax.ShapeDtypeStruct((n, m), jnp.float32),
    )(x, w)


def _relu_mm_kernel(x_ref, b_ref, w_ref, o_ref):
    h = jnp.maximum(x_ref[...] + b_ref[...], 0.0)
    o_ref[...] = jnp.dot(h, w_ref[...], preferred_element_type=jnp.float32)


def _relu_matmul(x, b, w):
    n, k = x.shape
    _, m = w.shape
    grid = pl.cdiv(n, ROW_BLK)
    return pl.pallas_call(
        _relu_mm_kernel,
        grid=(grid,),
        in_specs=[
            pl.BlockSpec((ROW_BLK, k), lambda i: (i, 0)),
            pl.BlockSpec((1, k), lambda i: (0, 0)),
            pl.BlockSpec((k, m), lambda i: (0, 0)),
        ],
        out_specs=pl.BlockSpec((ROW_BLK, m), lambda i: (i, 0)),
        out_shape=jax.ShapeDtypeStruct((n, m), jnp.float32),
    )(x, b.reshape(1, k), w)


def _tail_kernel(x_ref, h_ref, b_ref, o_ref):
    o_ref[...] = jnp.tanh(x_ref[:, :128] + h_ref[...] + b_ref[...])


def _tail(x, h, b):
    n = N_NODES
    m = 128
    blk = 1000
    return pl.pallas_call(
        _tail_kernel,
        grid=(n // blk,),
        in_specs=[
            pl.BlockSpec((blk, x.shape[1]), lambda i: (i, 0)),
            pl.BlockSpec((blk, m), lambda i: (i, 0)),
            pl.BlockSpec((1, m), lambda i: (0, 0)),
        ],
        out_specs=pl.BlockSpec((blk, m), lambda i: (i, 0)),
        out_shape=jax.ShapeDtypeStruct((n, m), jnp.float32),
    )(x, h, b.reshape(1, m))


def kernel(x, edge_index, W1, b1, W2, b2):
    xp = jnp.pad(x, ((0, NPAD - N_NODES), (0, 0)))
    qsrc, qdst, counts = _bucket_edges(edge_index.reshape(-1))
    ssrc, gd, tot = _merge_sort(qsrc, qdst, counts)
    h1 = _matmul1(xp, W1)
    m1 = _segmax2(h1, ssrc, gd, tot)
    h2 = _relu_matmul(m1, b1, W2)
    m2 = _segmax2(h2, ssrc, gd, tot)
    return _tail(x, m2, b2)
